# Initial kernel scaffold; baseline (speedup 1.0000x reference)
import jax
import jax.numpy as jnp
from jax.experimental import pallas as pl


def kernel(coords, conn, prop_E, prop_A, prop_I22, elem_lengths, elem_directions, elem_load, bc_disp, bc_rot, W1, b1, W2, b2):
    raise NotImplementedError("write your pallas kernel here")


# speedup vs baseline: 5.4243x; 5.4243x over previous
import functools

import jax
import jax.numpy as jnp
from jax import lax
from jax.experimental import pallas as pl
from jax.experimental.pallas import tpu as pltpu
from jax.experimental.pallas import tpu_sc as plsc

N = 100000
E = 1600000
H = 64

BN = 1000
NB = N // BN

NC = 2
NS = 16
EPC = E // NC
EPT = EPC // NS
CH = 400
NCHUNK = EPT // CH
GRP = CH // 16
GB = 100
NGB = 2 * CH // GB
ZB = 112
WB = 784
SEG = 6272
NPAD = NS * SEG

W_FORCE = 1.0
W_MOMENT = 1.0
W_KIN = 0.1
W_NEUMANN = 1.0

f32 = jnp.float32
i32 = jnp.int32


def _node_body(c_ref, w1_ref, b1_ref, w2_ref, b2_ref, pred_ref, tab_ref):
    c = c_ref[...]
    w1 = w1_ref[...]
    b1 = b1_ref[...]
    w2 = w2_ref[...]
    b2 = b2_ref[...]
    z = jnp.dot(c, w1, preferred_element_type=f32) + b1[None, :]
    h = jnp.tanh(z)
    pred = jnp.dot(h, w2, preferred_element_type=f32) + b2[None, :]
    s = 1.0 - h * h
    dn = (((1,), (1,)), ((), ()))
    g0 = lax.dot_general(s * w2[:, 0][None, :], w1, dn, preferred_element_type=f32)
    g1 = lax.dot_general(s * w2[:, 1][None, :], w1, dn, preferred_element_type=f32)
    g2 = lax.dot_general(s * w2[:, 2][None, :], w1, dn, preferred_element_type=f32)
    pred_ref[...] = pred
    tab_ref[:, 0:3] = g0
    tab_ref[:, 3:6] = g1
    tab_ref[:, 6:9] = g2
    tab_ref[:, 9:10] = pred[:, 2:3]
    tab_ref[:, 10:16] = jnp.zeros((BN, 6), f32)


def _node_stage(coords, W1, b1, W2, b2):
    return pl.pallas_call(
        _node_body,
        grid=(NB,),
        in_specs=[
            pl.BlockSpec((BN, 3), lambda i: (i, 0)),
            pl.BlockSpec((3, H), lambda i: (0, 0)),
            pl.BlockSpec((H,), lambda i: (0,)),
            pl.BlockSpec((H, 3), lambda i: (0, 0)),
            pl.BlockSpec((3,), lambda i: (0,)),
        ],
        out_specs=[
            pl.BlockSpec((BN, 3), lambda i: (i, 0)),
            pl.BlockSpec((BN, 16), lambda i: (i, 0)),
        ],
        out_shape=[
            jax.ShapeDtypeStruct((N, 3), f32),
            jax.ShapeDtypeStruct((N, 16), f32),
        ],
    )(coords, W1, b1, W2, b2)


def _rsqrt16(x):
    i = lax.bitcast_convert_type(x, i32)
    y = lax.bitcast_convert_type(jnp.int32(0x5F3759DF) - (i >> 1), f32)
    for _ in range(3):
        y = y * (1.5 - 0.5 * x * y * y)
    return y


def _elem_body(tab_hbm, conn_hbm, pE_hbm, pA_hbm, pI_hbm, L_hbm, dir_hbm, q_hbm,
               acc_out, scal_out,
               idx_v, rows_v, pEv, pAv, pIv, Lv, dirv, qv, zbuf, red_v,
               acc_sh, sem):
    c = lax.axis_index("c")
    s = lax.axis_index("s")
    zeros16 = jnp.zeros((16,), f32)
    lane = lax.iota(i32, 16)

    def _zrow(r, _):
        zbuf[r, :] = zeros16
        return 0

    lax.fori_loop(0, ZB, _zrow, 0)
    segbase = s * SEG

    def _zseg(t, _):
        base = pl.multiple_of(segbase + t * ZB, ZB)
        pltpu.sync_copy(zbuf, acc_sh.at[pl.ds(base, ZB)])
        return 0

    lax.fori_loop(0, SEG // ZB, _zseg, 0)
    for k in range(4):
        red_v[k, :] = zeros16
    plsc.subcore_barrier()

    ebase = c * EPC + s * EPT

    def chunk_body(t, _):
        off = pl.multiple_of(ebase + t * CH, CH)
        crow = pl.multiple_of(off // (GB // 2), NGB)
        pltpu.sync_copy(conn_hbm.at[pl.ds(crow, NGB)], idx_v)
        pltpu.sync_copy(pE_hbm.at[pl.ds(off, CH)], pEv)
        pltpu.sync_copy(pA_hbm.at[pl.ds(off, CH)], pAv)
        pltpu.sync_copy(pI_hbm.at[pl.ds(off, CH)], pIv)
        pltpu.sync_copy(L_hbm.at[pl.ds(off, CH)], Lv)
        off3 = pl.multiple_of(3 * off, 3 * CH)
        pltpu.sync_copy(dir_hbm.at[pl.ds(off3, 3 * CH)], dirv)
        pltpu.sync_copy(q_hbm.at[pl.ds(off3, 3 * CH)], qv)
        cps = [
            pltpu.async_copy(tab_hbm.at[idx_v.at[j]],
                             rows_v.at[pl.ds(j * GB, GB)], sem)
            for j in range(NGB)
        ]
        for cp in cps:
            cp.wait()

        def group(g, _):
            e16 = g * 16 + lane
            ri = e16 * 2
            rj = ri + 1

            def col(k):
                return jnp.full((16,), k, i32)

            def ld(r, k):
                return plsc.load_gather(rows_v, [r, col(k)])

            gxi0, gxi1, gxi2 = ld(ri, 0), ld(ri, 1), ld(ri, 2)
            gzi0, gzi1, gzi2 = ld(ri, 3), ld(ri, 4), ld(ri, 5)
            gpi0, gpi1, gpi2 = ld(ri, 6), ld(ri, 7), ld(ri, 8)
            phi_i = ld(ri, 9)
            gxj0, gxj1, gxj2 = ld(rj, 0), ld(rj, 1), ld(rj, 2)
            gzj0, gzj1, gzj2 = ld(rj, 3), ld(rj, 4), ld(rj, 5)
            gpj0, gpj1, gpj2 = ld(rj, 6), ld(rj, 7), ld(rj, 8)
            phi_j = ld(rj, 9)
            e3 = e16 * 3
            xh0 = plsc.load_gather(dirv, [e3])
            xh1 = plsc.load_gather(dirv, [e3 + 1])
            xh2 = plsc.load_gather(dirv, [e3 + 2])
            q0 = plsc.load_gather(qv, [e3])
            q1 = plsc.load_gather(qv, [e3 + 1])
            q2 = plsc.load_gather(qv, [e3 + 2])
            gs = pl.multiple_of(g * 16, 16)
            pEl = pEv[pl.ds(gs, 16)]
            pAl = pAv[pl.ds(gs, 16)]
            pIl = pIv[pl.ds(gs, 16)]
            Ll = Lv[pl.ds(gs, 16)]

            par = jnp.abs(xh1) > 0.99
            z0 = jnp.where(par, xh1, -xh2)
            z1 = jnp.where(par, -xh0, zeros16)
            z2 = jnp.where(par, zeros16, xh0)
            zz = z0 * z0 + z1 * z1 + z2 * z2
            inv = jnp.minimum(_rsqrt16(zz), 1e8)
            z0, z1, z2 = z0 * inv, z1 * inv, z2 * inv
            y0 = z1 * xh2 - z2 * xh1
            y1 = z2 * xh0 - z0 * xh2
            y2 = z0 * xh1 - z1 * xh0
            yy = y0 * y0 + y1 * y1 + y2 * y2
            invy = jnp.minimum(_rsqrt16(yy), 1e8)
            y0, y1, y2 = y0 * invy, y1 * invy, y2 * invy

            dotxi = gxi0 * xh0 + gxi1 * xh1 + gxi2 * xh2
            dotzi = gzi0 * xh0 + gzi1 * xh1 + gzi2 * xh2
            dotxj = gxj0 * xh0 + gxj1 * xh1 + gxj2 * xh2
            dotzj = gzj0 * xh0 + gzj1 * xh1 + gzj2 * xh2
            eps_i = xh0 * dotxi + xh2 * dotzi
            eps_j = xh0 * dotxj + xh2 * dotzj
            kap_i = gpi0 * xh0 + gpi1 * xh1 + gpi2 * xh2
            kap_j = gpj0 * xh0 + gpj1 * xh1 + gpj2 * xh2
            EA = pEl * pAl
            EI = pEl * pIl
            N_avg = 0.5 * EA * (eps_i + eps_j)
            M_i = EI * kap_i
            M_j = EI * kap_j
            V = (M_j - M_i) / Ll
            Fi0 = N_avg * xh0 + V * z0
            Fi1 = N_avg * xh1 + V * z1
            Fi2 = N_avg * xh2 + V * z2
            Fe0 = q0 * Ll * 0.5
            Fe1 = q1 * Ll * 0.5
            Fe2 = q2 * Ll * 0.5

            du_i = z0 * dotxi + z2 * dotzi
            du_j = z0 * dotxj + z2 * dotzj
            rk_i = phi_i - du_i
            rk_j = phi_j - du_j
            red_v[0, :] = red_v[0, :] + rk_i * rk_i + rk_j * rk_j
            red_v[1, :] = red_v[1, :] + Ll
            red_v[2, :] = jnp.maximum(red_v[2, :], Ll)
            qm = jnp.maximum(jnp.abs(q0), jnp.abs(q1))
            qm = jnp.maximum(qm, jnp.abs(q2))
            red_v[3, :] = jnp.maximum(red_v[3, :], qm)

            def st(r, k, v):
                plsc.store_scatter(rows_v, [r, col(k)], v)

            st(ri, 0, Fi0)
            st(ri, 1, Fi1)
            st(ri, 2, Fi2)
            st(ri, 3, M_i * y0)
            st(ri, 4, M_i * y1)
            st(ri, 5, M_i * y2)
            st(ri, 6, Fe0)
            st(ri, 7, Fe1)
            st(ri, 8, Fe2)
            st(rj, 0, -Fi0)
            st(rj, 1, -Fi1)
            st(rj, 2, -Fi2)
            st(rj, 3, M_j * y0)
            st(rj, 4, M_j * y1)
            st(rj, 5, M_j * y2)
            st(rj, 6, Fe0)
            st(rj, 7, Fe1)
            st(rj, 8, Fe2)
            return 0

        lax.fori_loop(0, GRP, group, 0)
        for j in range(NGB):
            pltpu.sync_copy(rows_v.at[pl.ds(j * GB, GB)],
                            acc_sh.at[idx_v.at[j]], add=True)
        return 0

    lax.fori_loop(0, NCHUNK, chunk_body, 0)

    w = s * NC + c
    pltpu.sync_copy(red_v, scal_out.at[w])

    plsc.subcore_barrier()

    def _wseg(t, _):
        base = pl.multiple_of(segbase + t * WB, WB)
        pltpu.sync_copy(acc_sh.at[pl.ds(base, WB)],
                        acc_out.at[c, pl.ds(base, WB)])
        return 0

    lax.fori_loop(0, SEG // WB, _wseg, 0)


def _elem_stage(tab, conn2d, pE, pA, pI, L, dirf, qf):
    mesh = plsc.VectorSubcoreMesh(core_axis_name="c", subcore_axis_name="s")
    f = functools.partial(
        pl.kernel,
        out_type=[
            jax.ShapeDtypeStruct((NC, NPAD, 16), f32),
            jax.ShapeDtypeStruct((NC * NS, 4, 16), f32),
        ],
        mesh=mesh,
        compiler_params=pltpu.CompilerParams(
            use_tc_tiling_on_sc=False, needs_layout_passes=False),
        scratch_types=[
            pltpu.VMEM((NGB, GB), i32),
            pltpu.VMEM((2 * CH, 16), f32),
            pltpu.VMEM((CH,), f32),
            pltpu.VMEM((CH,), f32),
            pltpu.VMEM((CH,), f32),
            pltpu.VMEM((CH,), f32),
            pltpu.VMEM((3 * CH,), f32),
            pltpu.VMEM((3 * CH,), f32),
            pltpu.VMEM((ZB, 16), f32),
            pltpu.VMEM((4, 16), f32),
            pltpu.VMEM_SHARED((NPAD, 16), f32),
            pltpu.SemaphoreType.DMA,
        ],
    )(_elem_body)
    return f(tab, conn2d, pE, pA, pI, L, dirf, qf)


def _reduce_body(acc_ref, bcd_ref, bcr_ref, scal_ref, out_ref, acc_s):
    i = pl.program_id(0)

    @pl.when(i == 0)
    def _():
        for k in range(8):
            acc_s[k] = 0.0

    a = acc_ref[0] + acc_ref[1]
    F_int = a[:, 0:3]
    M_int = a[:, 3:6]
    F_ext = a[:, 6:9]
    bd = bcd_ref[...][:, 0]
    br = bcr_ref[...][:, 0]
    free_d = (bd < 0.5).astype(f32)
    free_r = (br < 0.5).astype(f32)
    pin = ((bd > 0.5) & (br < 0.5)).astype(f32)
    m2 = jnp.sum(M_int * M_int, axis=1)
    acc_s[0] = acc_s[0] + jnp.sum(jnp.sum((F_int + F_ext) ** 2, axis=1) * free_d)
    acc_s[1] = acc_s[1] + jnp.sum(jnp.sum(F_ext * F_ext, axis=1) * free_d)
    acc_s[2] = acc_s[2] + jnp.sum(m2 * free_r)
    acc_s[3] = acc_s[3] + jnp.sum(m2 * pin)
    acc_s[4] = acc_s[4] + jnp.sum(free_d)
    acc_s[5] = acc_s[5] + jnp.sum(free_r)
    acc_s[6] = acc_s[6] + jnp.sum(pin)

    @pl.when(i == NB - 1)
    def _():
        sc = scal_ref[...]
        kin_tot = jnp.sum(sc[:, 0, :])
        l_sum = jnp.sum(sc[:, 1, :])
        l_max = jnp.max(sc[:, 2, :])
        q_max = jnp.max(sc[:, 3, :])
        nd = jnp.maximum(acc_s[4] * 3.0, 1.0)
        nr = jnp.maximum(acc_s[5] * 3.0, 1.0)
        npin = jnp.maximum(acc_s[6] * 3.0, 1.0)
        F_char = jnp.maximum(jnp.sqrt(acc_s[1] / nd), 1.0)
        M_char = jnp.maximum(jnp.maximum(q_max, 1.0) * l_max * l_sum / 8.0, 1.0)
        L_force = acc_s[0] / (F_char * F_char) / nd
        L_moment = acc_s[2] / (M_char * M_char) / nr
        L_neumann = acc_s[3] / (M_char * M_char) / npin
        L_kin = 0.5 * kin_tot / float(E)
        total = (W_FORCE * L_force + W_MOMENT * L_moment
                 + W_NEUMANN * L_neumann + W_KIN * L_kin)
        out_ref[...] = jnp.reshape(total, (1, 1))


def _reduce_stage(acc, scal, bc_disp, bc_rot):
    return pl.pallas_call(
        _reduce_body,
        grid=(NB,),
        in_specs=[
            pl.BlockSpec((NC, BN, 16), lambda i: (0, i, 0)),
            pl.BlockSpec((BN, 1), lambda i: (i, 0)),
            pl.BlockSpec((BN, 1), lambda i: (i, 0)),
            pl.BlockSpec((NC * NS, 4, 16), lambda i: (0, 0, 0)),
        ],
        out_specs=pl.BlockSpec((1, 1), lambda i: (0, 0)),
        out_shape=jax.ShapeDtypeStruct((1, 1), f32),
        scratch_shapes=[pltpu.SMEM((8,), f32)],
    )(acc, bc_disp, bc_rot, scal)


def kernel(coords, conn, prop_E, prop_A, prop_I22, elem_lengths, elem_directions,
           elem_load, bc_disp, bc_rot, W1, b1, W2, b2):
    pred, tab = _node_stage(coords, W1, b1, W2, b2)
    conn2d = conn.astype(i32).reshape(2 * E // GB, GB)
    dirf = elem_directions.reshape(3 * E)
    qf = elem_load.reshape(3 * E)
    acc, scal = _elem_stage(tab, conn2d, prop_E, prop_A, prop_I22,
                            elem_lengths, dirf, qf)
    total = _reduce_stage(acc, scal, bc_disp, bc_rot)
    return total.reshape(()), pred


# speedup vs baseline: 6.0656x; 1.1182x over previous
import functools

import jax
import jax.numpy as jnp
from jax import lax
from jax.experimental import pallas as pl
from jax.experimental.pallas import tpu as pltpu
from jax.experimental.pallas import tpu_sc as plsc

N = 100000
E = 1600000
H = 64

BN = 1000
NB = N // BN

NC = 2
NS = 16
EPC = E // NC
EPT = EPC // NS
CH = 400
NCHUNK = EPT // CH
GRP = CH // 16
GB = 100
NGB = 2 * CH // GB
ZB = 224
WB = 784
SEG = 6272
BE = 512
NBE = E // BE
NPAD = NS * SEG

W_FORCE = 1.0
W_MOMENT = 1.0
W_KIN = 0.1
W_NEUMANN = 1.0

f32 = jnp.float32
i32 = jnp.int32


def _node_body(c_ref, w1_ref, b1_ref, w2_ref, b2_ref, pred_ref, tab_ref):
    c = c_ref[...]
    w1 = w1_ref[...]
    b1 = b1_ref[...]
    w2 = w2_ref[...]
    b2 = b2_ref[...]
    z = jnp.dot(c, w1, preferred_element_type=f32) + b1[None, :]
    h = jnp.tanh(z)
    pred = jnp.dot(h, w2, preferred_element_type=f32) + b2[None, :]
    s = 1.0 - h * h
    dn = (((1,), (1,)), ((), ()))
    g0 = lax.dot_general(s * w2[:, 0][None, :], w1, dn, preferred_element_type=f32)
    g1 = lax.dot_general(s * w2[:, 1][None, :], w1, dn, preferred_element_type=f32)
    g2 = lax.dot_general(s * w2[:, 2][None, :], w1, dn, preferred_element_type=f32)
    pred_ref[...] = pred
    tab_ref[:, 0:3] = g0
    tab_ref[:, 3:6] = g1
    tab_ref[:, 6:9] = g2
    tab_ref[:, 9:10] = pred[:, 2:3]
    tab_ref[:, 10:16] = jnp.zeros((BN, 6), f32)


def _node_stage(coords, W1, b1, W2, b2):
    return pl.pallas_call(
        _node_body,
        grid=(NB,),
        in_specs=[
            pl.BlockSpec((BN, 3), lambda i: (i, 0)),
            pl.BlockSpec((3, H), lambda i: (0, 0)),
            pl.BlockSpec((H,), lambda i: (0,)),
            pl.BlockSpec((H, 3), lambda i: (0, 0)),
            pl.BlockSpec((3,), lambda i: (0,)),
        ],
        out_specs=[
            pl.BlockSpec((BN, 3), lambda i: (i, 0)),
            pl.BlockSpec((BN, 16), lambda i: (i, 0)),
        ],
        out_shape=[
            jax.ShapeDtypeStruct((N, 3), f32),
            jax.ShapeDtypeStruct((N, 16), f32),
        ],
    )(coords, W1, b1, W2, b2)


def _prep_body(pE_ref, pA_ref, pI_ref, L_ref, dir_ref, q_ref,
               eah_ref, ei_ref, eil_ref,
               x0_ref, x1_ref, x2_ref, z0_ref, z1_ref, z2_ref,
               y0_ref, y1_ref, y2_ref, f0_ref, f1_ref, f2_ref, es_ref,
               acc_s):
    i = pl.program_id(0)
    pE = pE_ref[...]
    pA = pA_ref[...]
    pI = pI_ref[...]
    L = L_ref[...]
    d = dir_ref[...]
    q = q_ref[...]
    d0 = d[:, 0]
    d1 = d[:, 1]
    d2 = d[:, 2]
    par = jnp.abs(d1) > 0.99
    zero = jnp.zeros_like(d0)
    z0 = jnp.where(par, d1, -d2)
    z1 = jnp.where(par, -d0, zero)
    z2 = jnp.where(par, zero, d0)
    zn = jnp.maximum(jnp.sqrt(z0 * z0 + z1 * z1 + z2 * z2), 1e-8)
    z0, z1, z2 = z0 / zn, z1 / zn, z2 / zn
    y0 = z1 * d2 - z2 * d1
    y1 = z2 * d0 - z0 * d2
    y2 = z0 * d1 - z1 * d0
    yn = jnp.maximum(jnp.sqrt(y0 * y0 + y1 * y1 + y2 * y2), 1e-8)
    y0, y1, y2 = y0 / yn, y1 / yn, y2 / yn
    EA = pE * pA
    EI = pE * pI
    eah_ref[...] = 0.5 * EA
    ei_ref[...] = EI
    eil_ref[...] = EI / L
    x0_ref[...] = d0
    x1_ref[...] = d1
    x2_ref[...] = d2
    z0_ref[...] = z0
    z1_ref[...] = z1
    z2_ref[...] = z2
    y0_ref[...] = y0
    y1_ref[...] = y1
    y2_ref[...] = y2
    f0_ref[...] = q[:, 0] * L * 0.5
    f1_ref[...] = q[:, 1] * L * 0.5
    f2_ref[...] = q[:, 2] * L * 0.5

    @pl.when(i == 0)
    def _():
        acc_s[0] = 0.0
        acc_s[1] = 0.0
        acc_s[2] = 0.0

    acc_s[0] = acc_s[0] + jnp.sum(L)
    acc_s[1] = jnp.maximum(acc_s[1], jnp.max(L))
    acc_s[2] = jnp.maximum(acc_s[2], jnp.max(jnp.abs(q)))

    @pl.when(i == NBE - 1)
    def _():
        es_ref[0] = acc_s[0]
        es_ref[1] = acc_s[1]
        es_ref[2] = acc_s[2]
        for k in range(3, 8):
            es_ref[k] = 0.0


def _prep_stage(pE, pA, pI, L, dirs, loads):
    ev = jax.ShapeDtypeStruct((E,), f32)
    return pl.pallas_call(
        _prep_body,
        grid=(NBE,),
        in_specs=[
            pl.BlockSpec((BE,), lambda i: (i,)),
            pl.BlockSpec((BE,), lambda i: (i,)),
            pl.BlockSpec((BE,), lambda i: (i,)),
            pl.BlockSpec((BE,), lambda i: (i,)),
            pl.BlockSpec((BE, 3), lambda i: (i, 0)),
            pl.BlockSpec((BE, 3), lambda i: (i, 0)),
        ],
        out_specs=[pl.BlockSpec((BE,), lambda i: (i,))] * 15
        + [pl.BlockSpec(memory_space=pltpu.MemorySpace.SMEM)],
        out_shape=[ev] * 15 + [jax.ShapeDtypeStruct((8,), f32)],
        scratch_shapes=[pltpu.SMEM((8,), f32)],
    )(pE, pA, pI, L, dirs, loads)


def _rsqrt16(x):
    i = lax.bitcast_convert_type(x, i32)
    y = lax.bitcast_convert_type(jnp.int32(0x5F3759DF) - (i >> 1), f32)
    for _ in range(3):
        y = y * (1.5 - 0.5 * x * y * y)
    return y


def _elem_body(tab_hbm, conn_hbm,
               eah_h, ei_h, eil_h, x0_h, x1_h, x2_h, z0_h, z1_h, z2_h,
               y0_h, y1_h, y2_h, f0_h, f1_h, f2_h,
               acc_out, scal_out,
               idx_v, rows_v, fld_v, zbuf, red_v,
               acc_sh, sem):
    c = lax.axis_index("c")
    s = lax.axis_index("s")
    zeros16 = jnp.zeros((16,), f32)
    lane = lax.iota(i32, 16)

    def _zrow(r, _):
        zbuf[r, :] = zeros16
        return 0

    lax.fori_loop(0, ZB, _zrow, 0)
    segbase = s * SEG

    def _zseg(t, _):
        base = pl.multiple_of(segbase + t * ZB, ZB)
        pltpu.sync_copy(zbuf, acc_sh.at[pl.ds(base, ZB)])
        return 0

    lax.fori_loop(0, SEG // ZB, _zseg, 0)
    red_v[0, :] = zeros16
    plsc.subcore_barrier()

    ebase = c * EPC + s * EPT
    fields = (eah_h, ei_h, eil_h, x0_h, x1_h, x2_h, z0_h, z1_h, z2_h,
              y0_h, y1_h, y2_h, f0_h, f1_h, f2_h)

    def chunk_body(t, _):
        off = pl.multiple_of(ebase + t * CH, CH)
        crow = pl.multiple_of(off // (GB // 2), NGB)
        pltpu.sync_copy(conn_hbm.at[pl.ds(crow, NGB)], idx_v)
        for k, fh in enumerate(fields):
            pltpu.sync_copy(fh.at[pl.ds(off, CH)], fld_v.at[k])
        cps = [
            pltpu.async_copy(tab_hbm.at[idx_v.at[j]],
                             rows_v.at[pl.ds(j * GB, GB)], sem)
            for j in range(NGB)
        ]
        for cp in cps:
            cp.wait()

        def group(g, _):
            e16 = g * 16 + lane
            ri = e16 * 2
            rj = ri + 1

            def col(k):
                return jnp.full((16,), k, i32)

            def ld(r, k):
                return plsc.load_gather(rows_v, [r, col(k)])

            gs = pl.multiple_of(g * 16, 16)

            def fl(k):
                return fld_v[k, pl.ds(gs, 16)]

            gxi0, gxi1, gxi2 = ld(ri, 0), ld(ri, 1), ld(ri, 2)
            gzi0, gzi1, gzi2 = ld(ri, 3), ld(ri, 4), ld(ri, 5)
            gpi0, gpi1, gpi2 = ld(ri, 6), ld(ri, 7), ld(ri, 8)
            phi_i = ld(ri, 9)
            gxj0, gxj1, gxj2 = ld(rj, 0), ld(rj, 1), ld(rj, 2)
            gzj0, gzj1, gzj2 = ld(rj, 3), ld(rj, 4), ld(rj, 5)
            gpj0, gpj1, gpj2 = ld(rj, 6), ld(rj, 7), ld(rj, 8)
            phi_j = ld(rj, 9)
            EAh = fl(0)
            EIe = fl(1)
            EIL = fl(2)
            xh0 = fl(3)
            xh1 = fl(4)
            xh2 = fl(5)
            z0 = fl(6)
            z1 = fl(7)
            z2 = fl(8)
            y0 = fl(9)
            y1 = fl(10)
            y2 = fl(11)
            Fe0 = fl(12)
            Fe1 = fl(13)
            Fe2 = fl(14)

            dotxi = gxi0 * xh0 + gxi1 * xh1 + gxi2 * xh2
            dotzi = gzi0 * xh0 + gzi1 * xh1 + gzi2 * xh2
            dotxj = gxj0 * xh0 + gxj1 * xh1 + gxj2 * xh2
            dotzj = gzj0 * xh0 + gzj1 * xh1 + gzj2 * xh2
            eps_i = xh0 * dotxi + xh2 * dotzi
            eps_j = xh0 * dotxj + xh2 * dotzj
            kap_i = gpi0 * xh0 + gpi1 * xh1 + gpi2 * xh2
            kap_j = gpj0 * xh0 + gpj1 * xh1 + gpj2 * xh2
            N_avg = EAh * (eps_i + eps_j)
            M_i = EIe * kap_i
            M_j = EIe * kap_j
            V = EIL * (kap_j - kap_i)
            Fi0 = N_avg * xh0 + V * z0
            Fi1 = N_avg * xh1 + V * z1
            Fi2 = N_avg * xh2 + V * z2

            du_i = z0 * dotxi + z2 * dotzi
            du_j = z0 * dotxj + z2 * dotzj
            rk_i = phi_i - du_i
            rk_j = phi_j - du_j
            red_v[0, :] = red_v[0, :] + rk_i * rk_i + rk_j * rk_j

            def st(r, k, v):
                plsc.store_scatter(rows_v, [r, col(k)], v)

            st(ri, 0, Fi0)
            st(ri, 1, Fi1)
            st(ri, 2, Fi2)
            st(ri, 3, M_i * y0)
            st(ri, 4, M_i * y1)
            st(ri, 5, M_i * y2)
            st(ri, 6, Fe0)
            st(ri, 7, Fe1)
            st(ri, 8, Fe2)
            st(rj, 0, -Fi0)
            st(rj, 1, -Fi1)
            st(rj, 2, -Fi2)
            st(rj, 3, M_j * y0)
            st(rj, 4, M_j * y1)
            st(rj, 5, M_j * y2)
            st(rj, 6, Fe0)
            st(rj, 7, Fe1)
            st(rj, 8, Fe2)
            return 0

        lax.fori_loop(0, GRP, group, 0)
        for j in range(NGB):
            pltpu.sync_copy(rows_v.at[pl.ds(j * GB, GB)],
                            acc_sh.at[idx_v.at[j]], add=True)
        return 0

    lax.fori_loop(0, NCHUNK, chunk_body, 0)

    w = s * NC + c
    pltpu.sync_copy(red_v, scal_out.at[w])

    plsc.subcore_barrier()

    def _wseg(t, _):
        base = pl.multiple_of(segbase + t * WB, WB)
        pltpu.sync_copy(acc_sh.at[pl.ds(base, WB)],
                        acc_out.at[c, pl.ds(base, WB)])
        return 0

    lax.fori_loop(0, SEG // WB, _wseg, 0)


def _elem_stage(tab, conn2d, prep):
    mesh = plsc.VectorSubcoreMesh(core_axis_name="c", subcore_axis_name="s")
    f = functools.partial(
        pl.kernel,
        out_type=[
            jax.ShapeDtypeStruct((NC, NPAD, 16), f32),
            jax.ShapeDtypeStruct((NC * NS, 1, 16), f32),
        ],
        mesh=mesh,
        compiler_params=pltpu.CompilerParams(
            use_tc_tiling_on_sc=False, needs_layout_passes=False),
        scratch_types=[
            pltpu.VMEM((NGB, GB), i32),
            pltpu.VMEM((2 * CH, 16), f32),
            pltpu.VMEM((15, CH), f32),
            pltpu.VMEM((ZB, 16), f32),
            pltpu.VMEM((1, 16), f32),
            pltpu.VMEM_SHARED((NPAD, 16), f32),
            pltpu.SemaphoreType.DMA,
        ],
    )(_elem_body)
    return f(tab, conn2d, *prep)


def _reduce_body(acc_ref, bcd_ref, bcr_ref, scal_ref, escal_ref, out_ref, acc_s):
    i = pl.program_id(0)

    @pl.when(i == 0)
    def _():
        for k in range(8):
            acc_s[k] = 0.0

    a = acc_ref[0] + acc_ref[1]
    F_int = a[:, 0:3]
    M_int = a[:, 3:6]
    F_ext = a[:, 6:9]
    bd = bcd_ref[...][:, 0]
    br = bcr_ref[...][:, 0]
    free_d = (bd < 0.5).astype(f32)
    free_r = (br < 0.5).astype(f32)
    pin = ((bd > 0.5) & (br < 0.5)).astype(f32)
    m2 = jnp.sum(M_int * M_int, axis=1)
    acc_s[0] = acc_s[0] + jnp.sum(jnp.sum((F_int + F_ext) ** 2, axis=1) * free_d)
    acc_s[1] = acc_s[1] + jnp.sum(jnp.sum(F_ext * F_ext, axis=1) * free_d)
    acc_s[2] = acc_s[2] + jnp.sum(m2 * free_r)
    acc_s[3] = acc_s[3] + jnp.sum(m2 * pin)
    acc_s[4] = acc_s[4] + jnp.sum(free_d)
    acc_s[5] = acc_s[5] + jnp.sum(free_r)
    acc_s[6] = acc_s[6] + jnp.sum(pin)

    @pl.when(i == NB - 1)
    def _():
        sc = scal_ref[...]
        kin_tot = jnp.sum(sc[:, 0, :])
        l_sum = escal_ref[0]
        l_max = escal_ref[1]
        q_max = escal_ref[2]
        nd = jnp.maximum(acc_s[4] * 3.0, 1.0)
        nr = jnp.maximum(acc_s[5] * 3.0, 1.0)
        npin = jnp.maximum(acc_s[6] * 3.0, 1.0)
        F_char = jnp.maximum(jnp.sqrt(acc_s[1] / nd), 1.0)
        M_char = jnp.maximum(jnp.maximum(q_max, 1.0) * l_max * l_sum / 8.0, 1.0)
        L_force = acc_s[0] / (F_char * F_char) / nd
        L_moment = acc_s[2] / (M_char * M_char) / nr
        L_neumann = acc_s[3] / (M_char * M_char) / npin
        L_kin = 0.5 * kin_tot / float(E)
        total = (W_FORCE * L_force + W_MOMENT * L_moment
                 + W_NEUMANN * L_neumann + W_KIN * L_kin)
        out_ref[...] = jnp.reshape(total, (1, 1))


def _reduce_stage(acc, scal, escal, bc_disp, bc_rot):
    return pl.pallas_call(
        _reduce_body,
        grid=(NB,),
        in_specs=[
            pl.BlockSpec((NC, BN, 16), lambda i: (0, i, 0)),
            pl.BlockSpec((BN, 1), lambda i: (i, 0)),
            pl.BlockSpec((BN, 1), lambda i: (i, 0)),
            pl.BlockSpec((NC * NS, 1, 16), lambda i: (0, 0, 0)),
            pl.BlockSpec(memory_space=pltpu.MemorySpace.SMEM),
        ],
        out_specs=pl.BlockSpec((1, 1), lambda i: (0, 0)),
        out_shape=jax.ShapeDtypeStruct((1, 1), f32),
        scratch_shapes=[pltpu.SMEM((8,), f32)],
    )(acc, bc_disp, bc_rot, scal, escal)


def kernel(coords, conn, prop_E, prop_A, prop_I22, elem_lengths, elem_directions,
           elem_load, bc_disp, bc_rot, W1, b1, W2, b2):
    pred, tab = _node_stage(coords, W1, b1, W2, b2)
    prep = _prep_stage(prop_E, prop_A, prop_I22, elem_lengths,
                       elem_directions, elem_load)
    conn2d = conn.astype(i32).reshape(2 * E // GB, GB)
    acc, scal = _elem_stage(tab, conn2d, prep[:15])
    total = _reduce_stage(acc, scal, prep[15], bc_disp, bc_rot)
    return total.reshape(()), pred


# speedup vs baseline: 21.4035x; 3.5287x over previous
import functools

import jax
import jax.numpy as jnp
from jax import lax
from jax.experimental import pallas as pl
from jax.experimental.pallas import tpu as pltpu
from jax.experimental.pallas import tpu_sc as plsc

N = 100000
E = 1600000
H = 64

NC = 2
NS = 16
EPC = E // NC
EPT = EPC // NS
CH = 400
NCHUNK = EPT // CH
GRP = CH // 16
GB = 100
NGB = 2 * CH // GB
ZB = 224
WB = 784
SEG = 6272
NPAD = NS * SEG
NPT = NPAD // (NC * NS)
RC = 224
BE = 8192
E2 = 196 * BE
NBE = E2 // BE
BN = 1024
NBN = NPAD // BN

W_FORCE = 1.0
W_MOMENT = 1.0
W_KIN = 0.1
W_NEUMANN = 1.0

f32 = jnp.float32
i32 = jnp.int32


def _node_body(c_ref, w1_ref, b1_ref, w2_ref, b2_ref, pred_ref, *col_refs):
    c = c_ref[...]
    w1 = w1_ref[...]
    b1 = b1_ref[...]
    w2 = w2_ref[...]
    b2 = b2_ref[...]
    z = jnp.dot(c, w1, preferred_element_type=f32) + b1[None, :]
    h = jnp.tanh(z)
    pred = jnp.dot(h, w2, preferred_element_type=f32) + b2[None, :]
    s = 1.0 - h * h
    dn = (((1,), (1,)), ((), ()))
    g0 = lax.dot_general(s * w2[:, 0][None, :], w1, dn, preferred_element_type=f32)
    g1 = lax.dot_general(s * w2[:, 1][None, :], w1, dn, preferred_element_type=f32)
    g2 = lax.dot_general(s * w2[:, 2][None, :], w1, dn, preferred_element_type=f32)
    pred_ref[...] = pred
    for k in range(3):
        col_refs[k][...] = g0[:, k]
        col_refs[3 + k][...] = g1[:, k]
        col_refs[6 + k][...] = g2[:, k]
    col_refs[9][...] = pred[:, 2]


def _node_stage(coords_p, W1, b1, W2, b2):
    nv = jax.ShapeDtypeStruct((NPAD,), f32)
    return pl.pallas_call(
        _node_body,
        grid=(NBN,),
        in_specs=[
            pl.BlockSpec((BN, 3), lambda i: (i, 0)),
            pl.BlockSpec((3, H), lambda i: (0, 0)),
            pl.BlockSpec((H,), lambda i: (0,)),
            pl.BlockSpec((H, 3), lambda i: (0, 0)),
            pl.BlockSpec((3,), lambda i: (0,)),
        ],
        out_specs=[pl.BlockSpec((BN, 3), lambda i: (i, 0))]
        + [pl.BlockSpec((BN,), lambda i: (i,))] * 10,
        out_shape=[jax.ShapeDtypeStruct((NPAD, 3), f32)] + [nv] * 10,
    )(coords_p, W1, b1, W2, b2)


def _prep_body(pE_ref, pA_ref, pI_ref, L_ref, d0_ref, d1_ref, d2_ref,
               q0_ref, q1_ref, q2_ref,
               eah_ref, ei_ref, eil_ref,
               x0_ref, x1_ref, x2_ref, z0_ref, z1_ref, z2_ref,
               y0_ref, y1_ref, y2_ref, f0_ref, f1_ref, f2_ref, es_ref,
               acc_s):
    i = pl.program_id(0)
    pE = pE_ref[...]
    pA = pA_ref[...]
    pI = pI_ref[...]
    L = L_ref[...]
    d0 = d0_ref[...]
    d1 = d1_ref[...]
    d2 = d2_ref[...]
    q0 = q0_ref[...]
    q1 = q1_ref[...]
    q2 = q2_ref[...]
    par = jnp.abs(d1) > 0.99
    zero = jnp.zeros_like(d0)
    z0 = jnp.where(par, d1, -d2)
    z1 = jnp.where(par, -d0, zero)
    z2 = jnp.where(par, zero, d0)
    zn = jnp.maximum(jnp.sqrt(z0 * z0 + z1 * z1 + z2 * z2), 1e-8)
    z0, z1, z2 = z0 / zn, z1 / zn, z2 / zn
    y0 = z1 * d2 - z2 * d1
    y1 = z2 * d0 - z0 * d2
    y2 = z0 * d1 - z1 * d0
    yn = jnp.maximum(jnp.sqrt(y0 * y0 + y1 * y1 + y2 * y2), 1e-8)
    y0, y1, y2 = y0 / yn, y1 / yn, y2 / yn
    EA = pE * pA
    EI = pE * pI
    eah_ref[...] = 0.5 * EA
    ei_ref[...] = EI
    eil_ref[...] = EI / L
    x0_ref[...] = d0
    x1_ref[...] = d1
    x2_ref[...] = d2
    z0_ref[...] = z0
    z1_ref[...] = z1
    z2_ref[...] = z2
    y0_ref[...] = y0
    y1_ref[...] = y1
    y2_ref[...] = y2
    f0_ref[...] = q0 * L * 0.5
    f1_ref[...] = q1 * L * 0.5
    f2_ref[...] = q2 * L * 0.5

    @pl.when(i == 0)
    def _():
        acc_s[0] = 0.0
        acc_s[1] = 0.0
        acc_s[2] = 0.0

    acc_s[0] = acc_s[0] + jnp.sum(L)
    acc_s[1] = jnp.maximum(acc_s[1], jnp.max(L))
    qm = jnp.maximum(jnp.max(jnp.abs(q0)), jnp.max(jnp.abs(q1)))
    acc_s[2] = jnp.maximum(acc_s[2], jnp.maximum(qm, jnp.max(jnp.abs(q2))))

    @pl.when(i == NBE - 1)
    def _():
        es_ref[0] = acc_s[0]
        es_ref[1] = acc_s[1]
        es_ref[2] = acc_s[2]
        for k in range(3, 8):
            es_ref[k] = 0.0


def _prep_stage(*cols):
    ev = jax.ShapeDtypeStruct((E2,), f32)
    return pl.pallas_call(
        _prep_body,
        grid=(NBE,),
        in_specs=[pl.BlockSpec((BE,), lambda i: (i,))] * 10,
        out_specs=[pl.BlockSpec((BE,), lambda i: (i,))] * 15
        + [pl.BlockSpec(memory_space=pltpu.MemorySpace.SMEM)],
        out_shape=[ev] * 15 + [jax.ShapeDtypeStruct((8,), f32)],
        scratch_shapes=[pltpu.SMEM((8,), f32)],
    )(*cols)


def _repack_body(*refs):
    cols = refs[:10]
    tab_out = refs[10]
    in_v = refs[11]
    out_v = refs[12]
    c = lax.axis_index("c")
    s = lax.axis_index("s")
    w = s * NC + c
    lane = lax.iota(i32, 16)
    zeros16 = jnp.zeros((16,), f32)

    def _zrow(r, _):
        out_v[r, :] = zeros16
        return 0

    lax.fori_loop(0, RC, _zrow, 0)
    base0 = w * NPT

    def chunk(t, _):
        base = pl.multiple_of(base0 + t * RC, RC)
        for k in range(10):
            pltpu.sync_copy(cols[k].at[pl.ds(base, RC)], in_v.at[k])

        def group(g, _):
            r = g * 16 + lane
            gs = pl.multiple_of(g * 16, 16)
            for k in range(10):
                plsc.store_scatter(out_v, [r, jnp.full((16,), k, i32)],
                                   in_v[k, pl.ds(gs, 16)])
            return 0

        lax.fori_loop(0, RC // 16, group, 0)
        pltpu.sync_copy(out_v, tab_out.at[pl.ds(base, RC)])
        return 0

    lax.fori_loop(0, NPT // RC, chunk, 0)


def _repack_stage(cols):
    mesh = plsc.VectorSubcoreMesh(core_axis_name="c", subcore_axis_name="s")
    f = functools.partial(
        pl.kernel,
        out_type=jax.ShapeDtypeStruct((NPAD, 16), f32),
        mesh=mesh,
        compiler_params=pltpu.CompilerParams(
            use_tc_tiling_on_sc=False, needs_layout_passes=False),
        scratch_types=[
            pltpu.VMEM((10, RC), f32),
            pltpu.VMEM((RC, 16), f32),
        ],
    )(_repack_body)
    return f(*cols)


def _elem_body(tab_hbm, conn_hbm,
               eah_h, ei_h, eil_h, x0_h, x1_h, x2_h, z0_h, z1_h, z2_h,
               y0_h, y1_h, y2_h, f0_h, f1_h, f2_h,
               acc_out, scal_out,
               idx_v, rows_v, fld_v, zbuf, red_v,
               acc_sh, sem, sem2):
    c = lax.axis_index("c")
    s = lax.axis_index("s")
    zeros16 = jnp.zeros((16,), f32)
    lane = lax.iota(i32, 16)

    def _zrow(r, _):
        zbuf[r, :] = zeros16
        return 0

    lax.fori_loop(0, ZB, _zrow, 0)
    segbase = s * SEG

    def _zseg(t, _):
        base = pl.multiple_of(segbase + t * ZB, ZB)
        pltpu.sync_copy(zbuf, acc_sh.at[pl.ds(base, ZB)])
        return 0

    lax.fori_loop(0, SEG // ZB, _zseg, 0)
    red_v[0, :] = zeros16
    plsc.subcore_barrier()

    ebase = c * EPC + s * EPT
    fields = (eah_h, ei_h, eil_h, x0_h, x1_h, x2_h, z0_h, z1_h, z2_h,
              y0_h, y1_h, y2_h, f0_h, f1_h, f2_h)

    def chunk_body(t, _):
        off = pl.multiple_of(ebase + t * CH, CH)
        crow = pl.multiple_of(off // (GB // 2), NGB)
        ccp = pltpu.async_copy(conn_hbm.at[pl.ds(crow, NGB)], idx_v, sem)
        fcps = [
            pltpu.async_copy(fh.at[pl.ds(off, CH)], fld_v.at[k], sem2)
            for k, fh in enumerate(fields)
        ]
        ccp.wait()
        cps = [
            pltpu.async_copy(tab_hbm.at[idx_v.at[j]],
                             rows_v.at[pl.ds(j * GB, GB)], sem)
            for j in range(NGB)
        ]
        for cp in fcps:
            cp.wait()
        for cp in cps:
            cp.wait()

        def group(g, _):
            e16 = g * 16 + lane
            ri = e16 * 2
            rj = ri + 1

            def col(k):
                return jnp.full((16,), k, i32)

            def ld(r, k):
                return plsc.load_gather(rows_v, [r, col(k)])

            gs = pl.multiple_of(g * 16, 16)

            def fl(k):
                return fld_v[k, pl.ds(gs, 16)]

            gxi0, gxi1, gxi2 = ld(ri, 0), ld(ri, 1), ld(ri, 2)
            gzi0, gzi1, gzi2 = ld(ri, 3), ld(ri, 4), ld(ri, 5)
            gpi0, gpi1, gpi2 = ld(ri, 6), ld(ri, 7), ld(ri, 8)
            phi_i = ld(ri, 9)
            gxj0, gxj1, gxj2 = ld(rj, 0), ld(rj, 1), ld(rj, 2)
            gzj0, gzj1, gzj2 = ld(rj, 3), ld(rj, 4), ld(rj, 5)
            gpj0, gpj1, gpj2 = ld(rj, 6), ld(rj, 7), ld(rj, 8)
            phi_j = ld(rj, 9)
            EAh = fl(0)
            EIe = fl(1)
            EIL = fl(2)
            xh0 = fl(3)
            xh1 = fl(4)
            xh2 = fl(5)
            z0 = fl(6)
            z1 = fl(7)
            z2 = fl(8)
            y0 = fl(9)
            y1 = fl(10)
            y2 = fl(11)
            Fe0 = fl(12)
            Fe1 = fl(13)
            Fe2 = fl(14)

            dotxi = gxi0 * xh0 + gxi1 * xh1 + gxi2 * xh2
            dotzi = gzi0 * xh0 + gzi1 * xh1 + gzi2 * xh2
            dotxj = gxj0 * xh0 + gxj1 * xh1 + gxj2 * xh2
            dotzj = gzj0 * xh0 + gzj1 * xh1 + gzj2 * xh2
            eps_i = xh0 * dotxi + xh2 * dotzi
            eps_j = xh0 * dotxj + xh2 * dotzj
            kap_i = gpi0 * xh0 + gpi1 * xh1 + gpi2 * xh2
            kap_j = gpj0 * xh0 + gpj1 * xh1 + gpj2 * xh2
            N_avg = EAh * (eps_i + eps_j)
            M_i = EIe * kap_i
            M_j = EIe * kap_j
            V = EIL * (kap_j - kap_i)
            Fi0 = N_avg * xh0 + V * z0
            Fi1 = N_avg * xh1 + V * z1
            Fi2 = N_avg * xh2 + V * z2

            du_i = z0 * dotxi + z2 * dotzi
            du_j = z0 * dotxj + z2 * dotzj
            rk_i = phi_i - du_i
            rk_j = phi_j - du_j
            red_v[0, :] = red_v[0, :] + rk_i * rk_i + rk_j * rk_j

            def st(r, k, v):
                plsc.store_scatter(rows_v, [r, col(k)], v)

            st(ri, 0, Fi0)
            st(ri, 1, Fi1)
            st(ri, 2, Fi2)
            st(ri, 3, M_i * y0)
            st(ri, 4, M_i * y1)
            st(ri, 5, M_i * y2)
            st(ri, 6, Fe0)
            st(ri, 7, Fe1)
            st(ri, 8, Fe2)
            st(rj, 0, -Fi0)
            st(rj, 1, -Fi1)
            st(rj, 2, -Fi2)
            st(rj, 3, M_j * y0)
            st(rj, 4, M_j * y1)
            st(rj, 5, M_j * y2)
            st(rj, 6, Fe0)
            st(rj, 7, Fe1)
            st(rj, 8, Fe2)
            return 0

        lax.fori_loop(0, GRP, group, 0)
        for j in range(NGB):
            pltpu.sync_copy(rows_v.at[pl.ds(j * GB, GB)],
                            acc_sh.at[idx_v.at[j]], add=True)
        return 0

    lax.fori_loop(0, NCHUNK, chunk_body, 0)

    w = s * NC + c
    pltpu.sync_copy(red_v, scal_out.at[w])

    plsc.subcore_barrier()

    def _wseg(t, _):
        base = pl.multiple_of(segbase + t * WB, WB)
        pltpu.sync_copy(acc_sh.at[pl.ds(base, WB)],
                        acc_out.at[c, pl.ds(base, WB)])
        return 0

    lax.fori_loop(0, SEG // WB, _wseg, 0)


def _elem_stage(tab, conn2d, prep):
    mesh = plsc.VectorSubcoreMesh(core_axis_name="c", subcore_axis_name="s")
    f = functools.partial(
        pl.kernel,
        out_type=[
            jax.ShapeDtypeStruct((NC, NPAD, 16), f32),
            jax.ShapeDtypeStruct((NC * NS, 1, 16), f32),
        ],
        mesh=mesh,
        compiler_params=pltpu.CompilerParams(
            use_tc_tiling_on_sc=False, needs_layout_passes=False),
        scratch_types=[
            pltpu.VMEM((NGB, GB), i32),
            pltpu.VMEM((2 * CH, 16), f32),
            pltpu.VMEM((15, CH), f32),
            pltpu.VMEM((ZB, 16), f32),
            pltpu.VMEM((1, 16), f32),
            pltpu.VMEM_SHARED((NPAD, 16), f32),
            pltpu.SemaphoreType.DMA,
            pltpu.SemaphoreType.DMA,
        ],
    )(_elem_body)
    return f(tab, conn2d, *prep)


def _nreduce_body(acc_h, bcd_h, bcr_h, part_out,
                  a0_v, a1_v, bcd_v, bcr_v, red_v):
    c = lax.axis_index("c")
    s = lax.axis_index("s")
    w = s * NC + c
    lane = lax.iota(i32, 16)
    zeros16 = jnp.zeros((16,), f32)
    for k in range(8):
        red_v[k, :] = zeros16
    base0 = w * NPT

    def chunk(t, _):
        base = pl.multiple_of(base0 + t * RC, RC)
        pltpu.sync_copy(acc_h.at[0, pl.ds(base, RC)], a0_v)
        pltpu.sync_copy(acc_h.at[1, pl.ds(base, RC)], a1_v)
        pltpu.sync_copy(bcd_h.at[pl.ds(base, RC)], bcd_v)
        pltpu.sync_copy(bcr_h.at[pl.ds(base, RC)], bcr_v)

        def group(g, _):
            r = g * 16 + lane
            gs = pl.multiple_of(g * 16, 16)

            def av(k):
                kk = jnp.full((16,), k, i32)
                return (plsc.load_gather(a0_v, [r, kk])
                        + plsc.load_gather(a1_v, [r, kk]))

            v0, v1, v2 = av(0), av(1), av(2)
            m0, m1, m2c = av(3), av(4), av(5)
            e0, e1, e2 = av(6), av(7), av(8)
            bd = bcd_v[pl.ds(gs, 16)]
            br = bcr_v[pl.ds(gs, 16)]
            ones = jnp.full((16,), 1.0, f32)
            free_d = jnp.where(bd < 0.5, ones, zeros16)
            free_r = jnp.where(br < 0.5, ones, zeros16)
            pin = jnp.where(bd > 0.5, free_r, zeros16)
            t0 = v0 + e0
            t1 = v1 + e1
            t2 = v2 + e2
            fif = t0 * t0 + t1 * t1 + t2 * t2
            fe2 = e0 * e0 + e1 * e1 + e2 * e2
            m2 = m0 * m0 + m1 * m1 + m2c * m2c
            red_v[0, :] = red_v[0, :] + fif * free_d
            red_v[1, :] = red_v[1, :] + fe2 * free_d
            red_v[2, :] = red_v[2, :] + m2 * free_r
            red_v[3, :] = red_v[3, :] + m2 * pin
            red_v[4, :] = red_v[4, :] + free_d
            red_v[5, :] = red_v[5, :] + free_r
            red_v[6, :] = red_v[6, :] + pin
            return 0

        lax.fori_loop(0, RC // 16, group, 0)
        return 0

    lax.fori_loop(0, NPT // RC, chunk, 0)
    pltpu.sync_copy(red_v, part_out.at[w])


def _nreduce_stage(acc, bcd_p, bcr_p):
    mesh = plsc.VectorSubcoreMesh(core_axis_name="c", subcore_axis_name="s")
    f = functools.partial(
        pl.kernel,
        out_type=jax.ShapeDtypeStruct((NC * NS, 8, 16), f32),
        mesh=mesh,
        compiler_params=pltpu.CompilerParams(
            use_tc_tiling_on_sc=False, needs_layout_passes=False),
        scratch_types=[
            pltpu.VMEM((RC, 16), f32),
            pltpu.VMEM((RC, 16), f32),
            pltpu.VMEM((RC,), f32),
            pltpu.VMEM((RC,), f32),
            pltpu.VMEM((8, 16), f32),
        ],
    )(_nreduce_body)
    return f(acc, bcd_p, bcr_p)


def _final_body(part_ref, scal_ref, escal_ref, out_ref):
    p = part_ref[...]
    kin_tot = jnp.sum(scal_ref[...])
    s_fif = jnp.sum(p[:, 0, :])
    s_fe2 = jnp.sum(p[:, 1, :])
    s_mr = jnp.sum(p[:, 2, :])
    s_mp = jnp.sum(p[:, 3, :])
    c_d = jnp.sum(p[:, 4, :])
    c_r = jnp.sum(p[:, 5, :])
    c_p = jnp.sum(p[:, 6, :])
    l_sum = escal_ref[0]
    l_max = escal_ref[1]
    q_max = escal_ref[2]
    nd = jnp.maximum(c_d * 3.0, 1.0)
    nr = jnp.maximum(c_r * 3.0, 1.0)
    npin = jnp.maximum(c_p * 3.0, 1.0)
    F_char = jnp.maximum(jnp.sqrt(s_fe2 / nd), 1.0)
    M_char = jnp.maximum(jnp.maximum(q_max, 1.0) * l_max * l_sum / 8.0, 1.0)
    L_force = s_fif / (F_char * F_char) / nd
    L_moment = s_mr / (M_char * M_char) / nr
    L_neumann = s_mp / (M_char * M_char) / npin
    L_kin = 0.5 * kin_tot / float(E)
    total = (W_FORCE * L_force + W_MOMENT * L_moment
             + W_NEUMANN * L_neumann + W_KIN * L_kin)
    out_ref[...] = jnp.reshape(total, (1, 1))


def _final_stage(partials, scal, escal):
    return pl.pallas_call(
        _final_body,
        in_specs=[
            pl.BlockSpec((NC * NS, 8, 16), lambda: (0, 0, 0)),
            pl.BlockSpec((NC * NS, 1, 16), lambda: (0, 0, 0)),
            pl.BlockSpec(memory_space=pltpu.MemorySpace.SMEM),
        ],
        out_specs=pl.BlockSpec((1, 1), lambda: (0, 0)),
        out_shape=jax.ShapeDtypeStruct((1, 1), f32),
    )(partials, scal, escal)


def kernel(coords, conn, prop_E, prop_A, prop_I22, elem_lengths, elem_directions,
           elem_load, bc_disp, bc_rot, W1, b1, W2, b2):
    coords_p = jnp.pad(coords, ((0, NPAD - N), (0, 0)))
    node_out = _node_stage(coords_p, W1, b1, W2, b2)
    pred = node_out[0][:N]
    tab = _repack_stage(node_out[1:])
    ep = E2 - E
    prep = _prep_stage(
        jnp.pad(prop_E, (0, ep)), jnp.pad(prop_A, (0, ep)),
        jnp.pad(prop_I22, (0, ep)), jnp.pad(elem_lengths, (0, ep)),
        jnp.pad(elem_directions[:, 0], (0, ep)),
        jnp.pad(elem_directions[:, 1], (0, ep)),
        jnp.pad(elem_directions[:, 2], (0, ep)),
        jnp.pad(elem_load[:, 0], (0, ep)),
        jnp.pad(elem_load[:, 1], (0, ep)),
        jnp.pad(elem_load[:, 2], (0, ep)))
    conn2d = conn.astype(i32).reshape(2 * E // GB, GB)
    acc, scal = _elem_stage(tab, conn2d, prep[:15])
    bcd_p = jnp.pad(bc_disp[:, 0], (0, NPAD - N), constant_values=1.0)
    bcr_p = jnp.pad(bc_rot[:, 0], (0, NPAD - N), constant_values=1.0)
    partials = _nreduce_stage(acc, bcd_p, bcr_p)
    total = _final_stage(partials, scal, prep[15])
    return total.reshape(()), pred


# speedup vs baseline: 52.3796x; 2.4472x over previous
import functools

import jax
import jax.numpy as jnp
from jax import lax
from jax.experimental import pallas as pl
from jax.experimental.pallas import tpu as pltpu
from jax.experimental.pallas import tpu_sc as plsc

N = 100000
E = 1600000
H = 64

NC = 2
NS = 16
EPC = E // NC
EPT = EPC // NS
CH = 400
NCHUNK = EPT // CH
GRP = CH // 16
GB = 80
NGB = CH // GB
ZB = 224
WB = 784
SEG = 6272
NPAD = NS * SEG
NPT = NPAD // (NC * NS)
RC = 224
BE = 8192
E2 = 196 * BE
NBE = E2 // BE
BN = 1024
NBN = NPAD // BN

W_FORCE = 1.0
W_MOMENT = 1.0
W_KIN = 0.1
W_NEUMANN = 1.0

f32 = jnp.float32
i32 = jnp.int32


def _node_body(c_ref, w1_ref, b1_ref, w2_ref, b2_ref, pred_ref, *col_refs):
    c = c_ref[...]
    w1 = w1_ref[...]
    b1 = b1_ref[...]
    w2 = w2_ref[...]
    b2 = b2_ref[...]
    z = jnp.dot(c, w1, preferred_element_type=f32) + b1[None, :]
    h = jnp.tanh(z)
    pred = jnp.dot(h, w2, preferred_element_type=f32) + b2[None, :]
    s = 1.0 - h * h
    dn = (((1,), (1,)), ((), ()))
    g0 = lax.dot_general(s * w2[:, 0][None, :], w1, dn, preferred_element_type=f32)
    g1 = lax.dot_general(s * w2[:, 1][None, :], w1, dn, preferred_element_type=f32)
    g2 = lax.dot_general(s * w2[:, 2][None, :], w1, dn, preferred_element_type=f32)
    pred_ref[...] = pred
    for k in range(3):
        col_refs[k][...] = g0[:, k]
        col_refs[3 + k][...] = g1[:, k]
        col_refs[6 + k][...] = g2[:, k]
    col_refs[9][...] = pred[:, 2]


def _node_stage(coords_p, W1, b1, W2, b2):
    nv = jax.ShapeDtypeStruct((NPAD,), f32)
    return pl.pallas_call(
        _node_body,
        grid=(NBN,),
        in_specs=[
            pl.BlockSpec((BN, 3), lambda i: (i, 0)),
            pl.BlockSpec((3, H), lambda i: (0, 0)),
            pl.BlockSpec((H,), lambda i: (0,)),
            pl.BlockSpec((H, 3), lambda i: (0, 0)),
            pl.BlockSpec((3,), lambda i: (0,)),
        ],
        out_specs=[pl.BlockSpec((BN, 3), lambda i: (i, 0))]
        + [pl.BlockSpec((BN,), lambda i: (i,))] * 10,
        out_shape=[jax.ShapeDtypeStruct((NPAD, 3), f32)] + [nv] * 10,
    )(coords_p, W1, b1, W2, b2)


def _prep_body(pE_ref, pA_ref, pI_ref, L_ref, d0_ref, d1_ref, d2_ref,
               q0_ref, q1_ref, q2_ref,
               eah_ref, ei_ref, eil_ref,
               x0_ref, x1_ref, x2_ref, z0_ref, z1_ref, z2_ref,
               y0_ref, y1_ref, y2_ref, f0_ref, f1_ref, f2_ref, es_ref,
               acc_s):
    i = pl.program_id(0)
    pE = pE_ref[...]
    pA = pA_ref[...]
    pI = pI_ref[...]
    L = L_ref[...]
    d0 = d0_ref[...]
    d1 = d1_ref[...]
    d2 = d2_ref[...]
    q0 = q0_ref[...]
    q1 = q1_ref[...]
    q2 = q2_ref[...]
    par = jnp.abs(d1) > 0.99
    zero = jnp.zeros_like(d0)
    z0 = jnp.where(par, d1, -d2)
    z1 = jnp.where(par, -d0, zero)
    z2 = jnp.where(par, zero, d0)
    zn = jnp.maximum(jnp.sqrt(z0 * z0 + z1 * z1 + z2 * z2), 1e-8)
    z0, z1, z2 = z0 / zn, z1 / zn, z2 / zn
    y0 = z1 * d2 - z2 * d1
    y1 = z2 * d0 - z0 * d2
    y2 = z0 * d1 - z1 * d0
    yn = jnp.maximum(jnp.sqrt(y0 * y0 + y1 * y1 + y2 * y2), 1e-8)
    y0, y1, y2 = y0 / yn, y1 / yn, y2 / yn
    EA = pE * pA
    EI = pE * pI
    eah_ref[...] = 0.5 * EA
    ei_ref[...] = EI
    eil_ref[...] = EI / L
    x0_ref[...] = d0
    x1_ref[...] = d1
    x2_ref[...] = d2
    z0_ref[...] = z0
    z1_ref[...] = z1
    z2_ref[...] = z2
    y0_ref[...] = y0
    y1_ref[...] = y1
    y2_ref[...] = y2
    f0_ref[...] = q0 * L * 0.5
    f1_ref[...] = q1 * L * 0.5
    f2_ref[...] = q2 * L * 0.5

    @pl.when(i == 0)
    def _():
        acc_s[0] = 0.0
        acc_s[1] = 0.0
        acc_s[2] = 0.0

    acc_s[0] = acc_s[0] + jnp.sum(L)
    acc_s[1] = jnp.maximum(acc_s[1], jnp.max(L))
    qm = jnp.maximum(jnp.max(jnp.abs(q0)), jnp.max(jnp.abs(q1)))
    acc_s[2] = jnp.maximum(acc_s[2], jnp.maximum(qm, jnp.max(jnp.abs(q2))))

    @pl.when(i == NBE - 1)
    def _():
        es_ref[0] = acc_s[0]
        es_ref[1] = acc_s[1]
        es_ref[2] = acc_s[2]
        for k in range(3, 8):
            es_ref[k] = 0.0


def _prep_stage(*cols):
    ev = jax.ShapeDtypeStruct((E2,), f32)
    return pl.pallas_call(
        _prep_body,
        grid=(NBE,),
        in_specs=[pl.BlockSpec((BE,), lambda i: (i,))] * 10,
        out_specs=[pl.BlockSpec((BE,), lambda i: (i,))] * 15
        + [pl.BlockSpec(memory_space=pltpu.MemorySpace.SMEM)],
        out_shape=[ev] * 15 + [jax.ShapeDtypeStruct((8,), f32)],
        scratch_shapes=[pltpu.SMEM((8,), f32)],
    )(*cols)


def _repack_body(*refs):
    cols = refs[:10]
    tab_out = refs[10]
    in_v = refs[11]
    out_v = refs[12]
    c = lax.axis_index("c")
    s = lax.axis_index("s")
    w = s * NC + c
    lane = lax.iota(i32, 16)
    zeros16 = jnp.zeros((16,), f32)

    def _zrow(r, _):
        out_v[r, :] = zeros16
        return 0

    lax.fori_loop(0, RC, _zrow, 0)
    base0 = w * NPT

    def chunk(t, _):
        base = pl.multiple_of(base0 + t * RC, RC)
        for k in range(10):
            pltpu.sync_copy(cols[k].at[pl.ds(base, RC)], in_v.at[k])

        def group(g, _):
            r = g * 16 + lane
            gs = pl.multiple_of(g * 16, 16)
            for k in range(10):
                plsc.store_scatter(out_v, [r, jnp.full((16,), k, i32)],
                                   in_v[k, pl.ds(gs, 16)])
            return 0

        lax.fori_loop(0, RC // 16, group, 0)
        pltpu.sync_copy(out_v, tab_out.at[pl.ds(base, RC)])
        return 0

    lax.fori_loop(0, NPT // RC, chunk, 0)


def _repack_stage(cols):
    mesh = plsc.VectorSubcoreMesh(core_axis_name="c", subcore_axis_name="s")
    f = functools.partial(
        pl.kernel,
        out_type=jax.ShapeDtypeStruct((NPAD, 16), f32),
        mesh=mesh,
        compiler_params=pltpu.CompilerParams(
            use_tc_tiling_on_sc=False, needs_layout_passes=False),
        scratch_types=[
            pltpu.VMEM((10, RC), f32),
            pltpu.VMEM((RC, 16), f32),
        ],
    )(_repack_body)
    return f(*cols)


def _elem_body(tab_hbm, ci_hbm, cj_hbm,
               eah_h, ei_h, eil_h, x0_h, x1_h, x2_h, z0_h, z1_h, z2_h,
               y0_h, y1_h, y2_h, f0_h, f1_h, f2_h,
               acc_out, scal_out,
               idx_vi, idx_vj, rows_vi, rows_vj, fld_v, zbuf, red_v,
               acc_sh, sem, sem2, sem3):
    c = lax.axis_index("c")
    s = lax.axis_index("s")
    zeros16 = jnp.zeros((16,), f32)
    lane = lax.iota(i32, 16)

    def _zrow(r, _):
        zbuf[r, :] = zeros16
        return 0

    lax.fori_loop(0, ZB, _zrow, 0)
    segbase = s * SEG

    def _zseg(t, _):
        base = pl.multiple_of(segbase + t * ZB, ZB)
        pltpu.sync_copy(zbuf, acc_sh.at[pl.ds(base, ZB)])
        return 0

    lax.fori_loop(0, SEG // ZB, _zseg, 0)
    red_v[0, :] = zeros16
    plsc.subcore_barrier()

    ebase = c * EPC + s * EPT
    fields = (eah_h, ei_h, eil_h, x0_h, x1_h, x2_h, z0_h, z1_h, z2_h,
              y0_h, y1_h, y2_h, f0_h, f1_h, f2_h)

    def chunk_body(t, _):
        off = pl.multiple_of(ebase + t * CH, CH)
        ccps = []
        for j in range(NGB):
            o = pl.multiple_of(off + j * GB, GB)
            ccps.append(pltpu.async_copy(ci_hbm.at[pl.ds(o, GB)],
                                         idx_vi.at[j], sem))
            ccps.append(pltpu.async_copy(cj_hbm.at[pl.ds(o, GB)],
                                         idx_vj.at[j], sem))
        fcps = [
            pltpu.async_copy(fh.at[pl.ds(off, CH)], fld_v.at[k], sem2)
            for k, fh in enumerate(fields)
        ]
        for cp in ccps:
            cp.wait()
        cps = []
        for j in range(NGB):
            cps.append(pltpu.async_copy(
                tab_hbm.at[idx_vi.at[j]],
                rows_vi.at[pl.ds(j * GB, GB)], sem))
            cps.append(pltpu.async_copy(
                tab_hbm.at[idx_vj.at[j]],
                rows_vj.at[pl.ds(j * GB, GB)], sem))
        for cp in fcps:
            cp.wait()
        for cp in cps:
            cp.wait()

        def group(g, _):
            r = g * 16 + lane

            def col(k):
                return jnp.full((16,), k, i32)

            def ldi(k):
                return plsc.load_gather(rows_vi, [r, col(k)])

            def ldj(k):
                return plsc.load_gather(rows_vj, [r, col(k)])

            gs = pl.multiple_of(g * 16, 16)

            def fl(k):
                return fld_v[k, pl.ds(gs, 16)]

            gxi0, gxi1, gxi2 = ldi(0), ldi(1), ldi(2)
            gzi0, gzi1, gzi2 = ldi(3), ldi(4), ldi(5)
            gpi0, gpi1, gpi2 = ldi(6), ldi(7), ldi(8)
            phi_i = ldi(9)
            gxj0, gxj1, gxj2 = ldj(0), ldj(1), ldj(2)
            gzj0, gzj1, gzj2 = ldj(3), ldj(4), ldj(5)
            gpj0, gpj1, gpj2 = ldj(6), ldj(7), ldj(8)
            phi_j = ldj(9)
            EAh = fl(0)
            EIe = fl(1)
            EIL = fl(2)
            xh0 = fl(3)
            xh1 = fl(4)
            xh2 = fl(5)
            z0 = fl(6)
            z1 = fl(7)
            z2 = fl(8)
            y0 = fl(9)
            y1 = fl(10)
            y2 = fl(11)
            Fe0 = fl(12)
            Fe1 = fl(13)
            Fe2 = fl(14)

            dotxi = gxi0 * xh0 + gxi1 * xh1 + gxi2 * xh2
            dotzi = gzi0 * xh0 + gzi1 * xh1 + gzi2 * xh2
            dotxj = gxj0 * xh0 + gxj1 * xh1 + gxj2 * xh2
            dotzj = gzj0 * xh0 + gzj1 * xh1 + gzj2 * xh2
            eps_i = xh0 * dotxi + xh2 * dotzi
            eps_j = xh0 * dotxj + xh2 * dotzj
            kap_i = gpi0 * xh0 + gpi1 * xh1 + gpi2 * xh2
            kap_j = gpj0 * xh0 + gpj1 * xh1 + gpj2 * xh2
            N_avg = EAh * (eps_i + eps_j)
            M_i = EIe * kap_i
            M_j = EIe * kap_j
            V = EIL * (kap_j - kap_i)
            Fi0 = N_avg * xh0 + V * z0
            Fi1 = N_avg * xh1 + V * z1
            Fi2 = N_avg * xh2 + V * z2

            du_i = z0 * dotxi + z2 * dotzi
            du_j = z0 * dotxj + z2 * dotzj
            rk_i = phi_i - du_i
            rk_j = phi_j - du_j
            red_v[0, :] = red_v[0, :] + rk_i * rk_i + rk_j * rk_j

            def sti(k, v):
                plsc.store_scatter(rows_vi, [r, col(k)], v)

            def stj(k, v):
                plsc.store_scatter(rows_vj, [r, col(k)], v)

            sti(0, Fi0)
            sti(1, Fi1)
            sti(2, Fi2)
            sti(3, M_i * y0)
            sti(4, M_i * y1)
            sti(5, M_i * y2)
            sti(6, Fe0)
            sti(7, Fe1)
            sti(8, Fe2)
            stj(0, -Fi0)
            stj(1, -Fi1)
            stj(2, -Fi2)
            stj(3, M_j * y0)
            stj(4, M_j * y1)
            stj(5, M_j * y2)
            stj(6, Fe0)
            stj(7, Fe1)
            stj(8, Fe2)
            return 0

        lax.fori_loop(0, GRP, group, 0)
        scps = []
        for j in range(NGB):
            scps.append(pltpu.async_copy(
                rows_vi.at[pl.ds(j * GB, GB)],
                acc_sh.at[idx_vi.at[j]], sem3, add=True))
            scps.append(pltpu.async_copy(
                rows_vj.at[pl.ds(j * GB, GB)],
                acc_sh.at[idx_vj.at[j]], sem3, add=True))
        for cp in scps:
            cp.wait()
        return 0

    lax.fori_loop(0, NCHUNK, chunk_body, 0)

    w = s * NC + c
    pltpu.sync_copy(red_v, scal_out.at[w])

    plsc.subcore_barrier()

    def _wseg(t, _):
        base = pl.multiple_of(segbase + t * WB, WB)
        pltpu.sync_copy(acc_sh.at[pl.ds(base, WB)],
                        acc_out.at[c, pl.ds(base, WB)])
        return 0

    lax.fori_loop(0, SEG // WB, _wseg, 0)


def _elem_stage(tab, conn_i, conn_j, prep):
    mesh = plsc.VectorSubcoreMesh(core_axis_name="c", subcore_axis_name="s")
    f = functools.partial(
        pl.kernel,
        out_type=[
            jax.ShapeDtypeStruct((NC, NPAD, 16), f32),
            jax.ShapeDtypeStruct((NC * NS, 1, 16), f32),
        ],
        mesh=mesh,
        compiler_params=pltpu.CompilerParams(
            use_tc_tiling_on_sc=False, needs_layout_passes=False),
        scratch_types=[
            pltpu.VMEM((NGB, GB), i32),
            pltpu.VMEM((NGB, GB), i32),
            pltpu.VMEM((CH, 16), f32),
            pltpu.VMEM((CH, 16), f32),
            pltpu.VMEM((15, CH), f32),
            pltpu.VMEM((ZB, 16), f32),
            pltpu.VMEM((1, 16), f32),
            pltpu.VMEM_SHARED((NPAD, 16), f32),
            pltpu.SemaphoreType.DMA,
            pltpu.SemaphoreType.DMA,
            pltpu.SemaphoreType.DMA,
        ],
    )(_elem_body)
    return f(tab, conn_i, conn_j, *prep)


def _nreduce_body(acc_h, bcd_h, bcr_h, part_out,
                  a0_v, a1_v, bcd_v, bcr_v, red_v):
    c = lax.axis_index("c")
    s = lax.axis_index("s")
    w = s * NC + c
    lane = lax.iota(i32, 16)
    zeros16 = jnp.zeros((16,), f32)
    for k in range(8):
        red_v[k, :] = zeros16
    base0 = w * NPT

    def chunk(t, _):
        base = pl.multiple_of(base0 + t * RC, RC)
        pltpu.sync_copy(acc_h.at[0, pl.ds(base, RC)], a0_v)
        pltpu.sync_copy(acc_h.at[1, pl.ds(base, RC)], a1_v)
        pltpu.sync_copy(bcd_h.at[pl.ds(base, RC)], bcd_v)
        pltpu.sync_copy(bcr_h.at[pl.ds(base, RC)], bcr_v)

        def group(g, _):
            r = g * 16 + lane
            gs = pl.multiple_of(g * 16, 16)

            def av(k):
                kk = jnp.full((16,), k, i32)
                return (plsc.load_gather(a0_v, [r, kk])
                        + plsc.load_gather(a1_v, [r, kk]))

            v0, v1, v2 = av(0), av(1), av(2)
            m0, m1, m2c = av(3), av(4), av(5)
            e0, e1, e2 = av(6), av(7), av(8)
            bd = bcd_v[pl.ds(gs, 16)]
            br = bcr_v[pl.ds(gs, 16)]
            ones = jnp.full((16,), 1.0, f32)
            free_d = jnp.where(bd < 0.5, ones, zeros16)
            free_r = jnp.where(br < 0.5, ones, zeros16)
            pin = jnp.where(bd > 0.5, free_r, zeros16)
            t0 = v0 + e0
            t1 = v1 + e1
            t2 = v2 + e2
            fif = t0 * t0 + t1 * t1 + t2 * t2
            fe2 = e0 * e0 + e1 * e1 + e2 * e2
            m2 = m0 * m0 + m1 * m1 + m2c * m2c
            red_v[0, :] = red_v[0, :] + fif * free_d
            red_v[1, :] = red_v[1, :] + fe2 * free_d
            red_v[2, :] = red_v[2, :] + m2 * free_r
            red_v[3, :] = red_v[3, :] + m2 * pin
            red_v[4, :] = red_v[4, :] + free_d
            red_v[5, :] = red_v[5, :] + free_r
            red_v[6, :] = red_v[6, :] + pin
            return 0

        lax.fori_loop(0, RC // 16, group, 0)
        return 0

    lax.fori_loop(0, NPT // RC, chunk, 0)
    pltpu.sync_copy(red_v, part_out.at[w])


def _nreduce_stage(acc, bcd_p, bcr_p):
    mesh = plsc.VectorSubcoreMesh(core_axis_name="c", subcore_axis_name="s")
    f = functools.partial(
        pl.kernel,
        out_type=jax.ShapeDtypeStruct((NC * NS, 8, 16), f32),
        mesh=mesh,
        compiler_params=pltpu.CompilerParams(
            use_tc_tiling_on_sc=False, needs_layout_passes=False),
        scratch_types=[
            pltpu.VMEM((RC, 16), f32),
            pltpu.VMEM((RC, 16), f32),
            pltpu.VMEM((RC,), f32),
            pltpu.VMEM((RC,), f32),
            pltpu.VMEM((8, 16), f32),
        ],
    )(_nreduce_body)
    return f(acc, bcd_p, bcr_p)


def _final_body(part_ref, scal_ref, escal_ref, out_ref):
    p = part_ref[...]
    kin_tot = jnp.sum(scal_ref[...])
    s_fif = jnp.sum(p[:, 0, :])
    s_fe2 = jnp.sum(p[:, 1, :])
    s_mr = jnp.sum(p[:, 2, :])
    s_mp = jnp.sum(p[:, 3, :])
    c_d = jnp.sum(p[:, 4, :])
    c_r = jnp.sum(p[:, 5, :])
    c_p = jnp.sum(p[:, 6, :])
    l_sum = escal_ref[0]
    l_max = escal_ref[1]
    q_max = escal_ref[2]
    nd = jnp.maximum(c_d * 3.0, 1.0)
    nr = jnp.maximum(c_r * 3.0, 1.0)
    npin = jnp.maximum(c_p * 3.0, 1.0)
    F_char = jnp.maximum(jnp.sqrt(s_fe2 / nd), 1.0)
    M_char = jnp.maximum(jnp.maximum(q_max, 1.0) * l_max * l_sum / 8.0, 1.0)
    L_force = s_fif / (F_char * F_char) / nd
    L_moment = s_mr / (M_char * M_char) / nr
    L_neumann = s_mp / (M_char * M_char) / npin
    L_kin = 0.5 * kin_tot / float(E)
    total = (W_FORCE * L_force + W_MOMENT * L_moment
             + W_NEUMANN * L_neumann + W_KIN * L_kin)
    out_ref[...] = jnp.reshape(total, (1, 1))


def _final_stage(partials, scal, escal):
    return pl.pallas_call(
        _final_body,
        in_specs=[
            pl.BlockSpec((NC * NS, 8, 16), lambda: (0, 0, 0)),
            pl.BlockSpec((NC * NS, 1, 16), lambda: (0, 0, 0)),
            pl.BlockSpec(memory_space=pltpu.MemorySpace.SMEM),
        ],
        out_specs=pl.BlockSpec((1, 1), lambda: (0, 0)),
        out_shape=jax.ShapeDtypeStruct((1, 1), f32),
    )(partials, scal, escal)


def kernel(coords, conn, prop_E, prop_A, prop_I22, elem_lengths, elem_directions,
           elem_load, bc_disp, bc_rot, W1, b1, W2, b2):
    coords_p = jnp.pad(coords, ((0, NPAD - N), (0, 0)))
    node_out = _node_stage(coords_p, W1, b1, W2, b2)
    pred = node_out[0][:N]
    tab = _repack_stage(node_out[1:])
    ep = E2 - E
    prep = _prep_stage(
        jnp.pad(prop_E, (0, ep)), jnp.pad(prop_A, (0, ep)),
        jnp.pad(prop_I22, (0, ep)), jnp.pad(elem_lengths, (0, ep)),
        jnp.pad(elem_directions[:, 0], (0, ep)),
        jnp.pad(elem_directions[:, 1], (0, ep)),
        jnp.pad(elem_directions[:, 2], (0, ep)),
        jnp.pad(elem_load[:, 0], (0, ep)),
        jnp.pad(elem_load[:, 1], (0, ep)),
        jnp.pad(elem_load[:, 2], (0, ep)))
    conn_i = conn[:, 0].astype(i32)
    conn_j = conn[:, 1].astype(i32)
    acc, scal = _elem_stage(tab, conn_i, conn_j, prep[:15])
    bcd_p = jnp.pad(bc_disp[:, 0], (0, NPAD - N), constant_values=1.0)
    bcr_p = jnp.pad(bc_rot[:, 0], (0, NPAD - N), constant_values=1.0)
    partials = _nreduce_stage(acc, bcd_p, bcr_p)
    total = _final_stage(partials, scal, prep[15])
    return total.reshape(()), pred


# speedup vs baseline: 59.3761x; 1.1336x over previous
import functools

import jax
import jax.numpy as jnp
from jax import lax
from jax.experimental import pallas as pl
from jax.experimental.pallas import tpu as pltpu
from jax.experimental.pallas import tpu_sc as plsc

N = 100000
E = 1600000
H = 64

NC = 2
NS = 16
EPC = E // NC
EPT = EPC // NS
CH = 400
NCHUNK = EPT // CH
GRP = CH // 16
GB = 80
NGB = CH // GB
ZB = 112
WB = 784
SEG = 6272
NPAD = NS * SEG
NPT = NPAD // (NC * NS)
RC = 224
BE = 8192
E2 = 196 * BE
NBE = E2 // BE
BN = 1024
NBN = NPAD // BN

W_FORCE = 1.0
W_MOMENT = 1.0
W_KIN = 0.1
W_NEUMANN = 1.0

f32 = jnp.float32
i32 = jnp.int32


def _node_body(c_ref, w1_ref, b1_ref, w2_ref, b2_ref, pred_ref, *col_refs):
    c = c_ref[...]
    w1 = w1_ref[...]
    b1 = b1_ref[...]
    w2 = w2_ref[...]
    b2 = b2_ref[...]
    z = jnp.dot(c, w1, preferred_element_type=f32) + b1[None, :]
    h = jnp.tanh(z)
    pred = jnp.dot(h, w2, preferred_element_type=f32) + b2[None, :]
    s = 1.0 - h * h
    dn = (((1,), (1,)), ((), ()))
    g0 = lax.dot_general(s * w2[:, 0][None, :], w1, dn, preferred_element_type=f32)
    g1 = lax.dot_general(s * w2[:, 1][None, :], w1, dn, preferred_element_type=f32)
    g2 = lax.dot_general(s * w2[:, 2][None, :], w1, dn, preferred_element_type=f32)
    pred_ref[...] = pred
    for k in range(3):
        col_refs[k][...] = g0[:, k]
        col_refs[3 + k][...] = g1[:, k]
        col_refs[6 + k][...] = g2[:, k]
    col_refs[9][...] = pred[:, 2]


def _node_stage(coords_p, W1, b1, W2, b2):
    nv = jax.ShapeDtypeStruct((NPAD,), f32)
    return pl.pallas_call(
        _node_body,
        grid=(NBN,),
        in_specs=[
            pl.BlockSpec((BN, 3), lambda i: (i, 0)),
            pl.BlockSpec((3, H), lambda i: (0, 0)),
            pl.BlockSpec((H,), lambda i: (0,)),
            pl.BlockSpec((H, 3), lambda i: (0, 0)),
            pl.BlockSpec((3,), lambda i: (0,)),
        ],
        out_specs=[pl.BlockSpec((BN, 3), lambda i: (i, 0))]
        + [pl.BlockSpec((BN,), lambda i: (i,))] * 10,
        out_shape=[jax.ShapeDtypeStruct((NPAD, 3), f32)] + [nv] * 10,
    )(coords_p, W1, b1, W2, b2)


def _prep_body(pE_ref, pA_ref, pI_ref, L_ref, d0_ref, d1_ref, d2_ref,
               q0_ref, q1_ref, q2_ref,
               eah_ref, ei_ref, eil_ref,
               x0_ref, x1_ref, x2_ref, z0_ref, z1_ref, z2_ref,
               y0_ref, y1_ref, y2_ref, f0_ref, f1_ref, f2_ref, es_ref,
               acc_s):
    i = pl.program_id(0)
    pE = pE_ref[...]
    pA = pA_ref[...]
    pI = pI_ref[...]
    L = L_ref[...]
    d0 = d0_ref[...]
    d1 = d1_ref[...]
    d2 = d2_ref[...]
    q0 = q0_ref[...]
    q1 = q1_ref[...]
    q2 = q2_ref[...]
    par = jnp.abs(d1) > 0.99
    zero = jnp.zeros_like(d0)
    z0 = jnp.where(par, d1, -d2)
    z1 = jnp.where(par, -d0, zero)
    z2 = jnp.where(par, zero, d0)
    zn = jnp.maximum(jnp.sqrt(z0 * z0 + z1 * z1 + z2 * z2), 1e-8)
    z0, z1, z2 = z0 / zn, z1 / zn, z2 / zn
    y0 = z1 * d2 - z2 * d1
    y1 = z2 * d0 - z0 * d2
    y2 = z0 * d1 - z1 * d0
    yn = jnp.maximum(jnp.sqrt(y0 * y0 + y1 * y1 + y2 * y2), 1e-8)
    y0, y1, y2 = y0 / yn, y1 / yn, y2 / yn
    EA = pE * pA
    EI = pE * pI
    eah_ref[...] = 0.5 * EA
    ei_ref[...] = EI
    eil_ref[...] = EI / L
    x0_ref[...] = d0
    x1_ref[...] = d1
    x2_ref[...] = d2
    z0_ref[...] = z0
    z1_ref[...] = z1
    z2_ref[...] = z2
    y0_ref[...] = y0
    y1_ref[...] = y1
    y2_ref[...] = y2
    f0_ref[...] = q0 * L * 0.5
    f1_ref[...] = q1 * L * 0.5
    f2_ref[...] = q2 * L * 0.5

    @pl.when(i == 0)
    def _():
        acc_s[0] = 0.0
        acc_s[1] = 0.0
        acc_s[2] = 0.0

    acc_s[0] = acc_s[0] + jnp.sum(L)
    acc_s[1] = jnp.maximum(acc_s[1], jnp.max(L))
    qm = jnp.maximum(jnp.max(jnp.abs(q0)), jnp.max(jnp.abs(q1)))
    acc_s[2] = jnp.maximum(acc_s[2], jnp.maximum(qm, jnp.max(jnp.abs(q2))))

    @pl.when(i == NBE - 1)
    def _():
        es_ref[0] = acc_s[0]
        es_ref[1] = acc_s[1]
        es_ref[2] = acc_s[2]
        for k in range(3, 8):
            es_ref[k] = 0.0


def _prep_stage(*cols):
    ev = jax.ShapeDtypeStruct((E2,), f32)
    return pl.pallas_call(
        _prep_body,
        grid=(NBE,),
        in_specs=[pl.BlockSpec((BE,), lambda i: (i,))] * 10,
        out_specs=[pl.BlockSpec((BE,), lambda i: (i,))] * 15
        + [pl.BlockSpec(memory_space=pltpu.MemorySpace.SMEM)],
        out_shape=[ev] * 15 + [jax.ShapeDtypeStruct((8,), f32)],
        scratch_shapes=[pltpu.SMEM((8,), f32)],
    )(*cols)


def _repack_body(*refs):
    cols = refs[:10]
    tab_out = refs[10]
    in_v = refs[11]
    out_v = refs[12]
    c = lax.axis_index("c")
    s = lax.axis_index("s")
    w = s * NC + c
    lane = lax.iota(i32, 16)
    zeros16 = jnp.zeros((16,), f32)

    def _zrow(r, _):
        out_v[r, :] = zeros16
        return 0

    lax.fori_loop(0, RC, _zrow, 0)
    base0 = w * NPT

    def chunk(t, _):
        base = pl.multiple_of(base0 + t * RC, RC)
        for k in range(10):
            pltpu.sync_copy(cols[k].at[pl.ds(base, RC)], in_v.at[k])

        def group(g, _):
            r = g * 16 + lane
            gs = pl.multiple_of(g * 16, 16)
            for k in range(10):
                plsc.store_scatter(out_v, [r, jnp.full((16,), k, i32)],
                                   in_v[k, pl.ds(gs, 16)])
            return 0

        lax.fori_loop(0, RC // 16, group, 0)
        pltpu.sync_copy(out_v, tab_out.at[pl.ds(base, RC)])
        return 0

    lax.fori_loop(0, NPT // RC, chunk, 0)


def _repack_stage(cols):
    mesh = plsc.VectorSubcoreMesh(core_axis_name="c", subcore_axis_name="s")
    f = functools.partial(
        pl.kernel,
        out_type=jax.ShapeDtypeStruct((NPAD, 16), f32),
        mesh=mesh,
        compiler_params=pltpu.CompilerParams(
            use_tc_tiling_on_sc=False, needs_layout_passes=False),
        scratch_types=[
            pltpu.VMEM((10, RC), f32),
            pltpu.VMEM((RC, 16), f32),
        ],
    )(_repack_body)
    return f(*cols)


def _elem_body(tab_hbm, ci_hbm, cj_hbm,
               eah_h, ei_h, eil_h, x0_h, x1_h, x2_h, z0_h, z1_h, z2_h,
               y0_h, y1_h, y2_h, f0_h, f1_h, f2_h,
               acc_out, scal_out,
               idx_vi, idx_vj, rows_vi, rows_vj, fld_v, zbuf, red_v,
               acc_sh, sem_c0, sem_c1, sem_f0, sem_f1, sem_s,
               sg0, sg1, sg2, sg3, sg4):
    c = lax.axis_index("c")
    s = lax.axis_index("s")
    zeros16 = jnp.zeros((16,), f32)
    lane = lax.iota(i32, 16)
    sem_c = (sem_c0, sem_c1)
    sem_f = (sem_f0, sem_f1)
    sem_g = (sg0, sg1, sg2, sg3, sg4)
    fields = (eah_h, ei_h, eil_h, x0_h, x1_h, x2_h, z0_h, z1_h, z2_h,
              y0_h, y1_h, y2_h, f0_h, f1_h, f2_h)

    def _zrow(r, _):
        zbuf[r, :] = zeros16
        return 0

    lax.fori_loop(0, ZB, _zrow, 0)
    segbase = s * SEG

    def _zseg(t, _):
        base = pl.multiple_of(segbase + t * ZB, ZB)
        pltpu.sync_copy(zbuf, acc_sh.at[pl.ds(base, ZB)])
        return 0

    lax.fori_loop(0, SEG // ZB, _zseg, 0)
    red_v[0, :] = zeros16
    plsc.subcore_barrier()

    ebase = c * EPC + s * EPT

    def _off(t):
        return pl.multiple_of(ebase + t * CH, CH)

    def _conn_descs(t, b):
        off = _off(t)
        ds_ = []
        for j in range(NGB):
            o = pl.multiple_of(off + j * GB, GB)
            ds_.append(pltpu.make_async_copy(ci_hbm.at[pl.ds(o, GB)],
                                             idx_vi.at[b, j], sem_c[b]))
            ds_.append(pltpu.make_async_copy(cj_hbm.at[pl.ds(o, GB)],
                                             idx_vj.at[b, j], sem_c[b]))
        return ds_

    def _field_descs(t, b):
        off = _off(t)
        return [pltpu.make_async_copy(fh.at[pl.ds(off, CH)],
                                      fld_v.at[b, k], sem_f[b])
                for k, fh in enumerate(fields)]

    def _scat_descs(b):
        ds_ = []
        for j in range(NGB):
            ds_.append(pltpu.make_async_copy(
                rows_vi.at[pl.ds(j * GB, GB)],
                acc_sh.at[idx_vi.at[b, j]], sem_s))
            ds_.append(pltpu.make_async_copy(
                rows_vj.at[pl.ds(j * GB, GB)],
                acc_sh.at[idx_vj.at[b, j]], sem_s))
        return ds_

    def _fire(descs):
        for d in descs:
            d.start()

    def _wait(descs):
        for d in descs:
            d.wait()

    def _compute_sub(j, b):
        def group(g, _):
            r = g * 16 + lane

            def col(k):
                return jnp.full((16,), k, i32)

            def ldi(k):
                return plsc.load_gather(rows_vi, [r, col(k)])

            def ldj(k):
                return plsc.load_gather(rows_vj, [r, col(k)])

            gs = pl.multiple_of(g * 16, 16)

            def fl(k):
                return fld_v[b, k, pl.ds(gs, 16)]

            gxi0, gxi1, gxi2 = ldi(0), ldi(1), ldi(2)
            gzi0, gzi1, gzi2 = ldi(3), ldi(4), ldi(5)
            gpi0, gpi1, gpi2 = ldi(6), ldi(7), ldi(8)
            phi_i = ldi(9)
            gxj0, gxj1, gxj2 = ldj(0), ldj(1), ldj(2)
            gzj0, gzj1, gzj2 = ldj(3), ldj(4), ldj(5)
            gpj0, gpj1, gpj2 = ldj(6), ldj(7), ldj(8)
            phi_j = ldj(9)
            EAh = fl(0)
            EIe = fl(1)
            EIL = fl(2)
            xh0 = fl(3)
            xh1 = fl(4)
            xh2 = fl(5)
            z0 = fl(6)
            z1 = fl(7)
            z2 = fl(8)
            y0 = fl(9)
            y1 = fl(10)
            y2 = fl(11)
            Fe0 = fl(12)
            Fe1 = fl(13)
            Fe2 = fl(14)

            dotxi = gxi0 * xh0 + gxi1 * xh1 + gxi2 * xh2
            dotzi = gzi0 * xh0 + gzi1 * xh1 + gzi2 * xh2
            dotxj = gxj0 * xh0 + gxj1 * xh1 + gxj2 * xh2
            dotzj = gzj0 * xh0 + gzj1 * xh1 + gzj2 * xh2
            eps_i = xh0 * dotxi + xh2 * dotzi
            eps_j = xh0 * dotxj + xh2 * dotzj
            kap_i = gpi0 * xh0 + gpi1 * xh1 + gpi2 * xh2
            kap_j = gpj0 * xh0 + gpj1 * xh1 + gpj2 * xh2
            N_avg = EAh * (eps_i + eps_j)
            M_i = EIe * kap_i
            M_j = EIe * kap_j
            V = EIL * (kap_j - kap_i)
            Fi0 = N_avg * xh0 + V * z0
            Fi1 = N_avg * xh1 + V * z1
            Fi2 = N_avg * xh2 + V * z2

            du_i = z0 * dotxi + z2 * dotzi
            du_j = z0 * dotxj + z2 * dotzj
            rk_i = phi_i - du_i
            rk_j = phi_j - du_j
            red_v[0, :] = red_v[0, :] + rk_i * rk_i + rk_j * rk_j

            def sti(k, v):
                plsc.store_scatter(rows_vi, [r, col(k)], v)

            def stj(k, v):
                plsc.store_scatter(rows_vj, [r, col(k)], v)

            sti(0, Fi0)
            sti(1, Fi1)
            sti(2, Fi2)
            sti(3, M_i * y0)
            sti(4, M_i * y1)
            sti(5, M_i * y2)
            sti(6, Fe0)
            sti(7, Fe1)
            sti(8, Fe2)
            stj(0, -Fi0)
            stj(1, -Fi1)
            stj(2, -Fi2)
            stj(3, M_j * y0)
            stj(4, M_j * y1)
            stj(5, M_j * y2)
            stj(6, Fe0)
            stj(7, Fe1)
            stj(8, Fe2)
            return 0

        lax.fori_loop(j * (GRP // NGB), (j + 1) * (GRP // NGB), group, 0)

    def _chunk(t, b, first, prefetch):
        _wait(_conn_descs(t, b))
        if not first:
            _wait(_scat_descs(1 - b))
        gds = []
        for j in range(NGB):
            gds.append(pltpu.make_async_copy(
                tab_hbm.at[idx_vi.at[b, j]],
                rows_vi.at[pl.ds(j * GB, GB)], sem_g[j]))
            gds.append(pltpu.make_async_copy(
                tab_hbm.at[idx_vj.at[b, j]],
                rows_vj.at[pl.ds(j * GB, GB)], sem_g[j]))
        _fire(gds)
        if prefetch:
            tn = t + 1
            _fire(_conn_descs(tn, 1 - b))
            _fire(_field_descs(tn, 1 - b))
        _wait(_field_descs(t, b))
        for j in range(NGB):
            gds[2 * j].wait()
            gds[2 * j + 1].wait()
            _compute_sub(j, b)
            sd_i = pltpu.make_async_copy(
                rows_vi.at[pl.ds(j * GB, GB)],
                acc_sh.at[idx_vi.at[b, j]], sem_s)
            sd_j = pltpu.make_async_copy(
                rows_vj.at[pl.ds(j * GB, GB)],
                acc_sh.at[idx_vj.at[b, j]], sem_s)
            sd_i.start(add=True)
            sd_j.start(add=True)

    _fire(_conn_descs(0, 0))
    _fire(_field_descs(0, 0))

    def pair(u, _):
        t0 = u * 2
        _chunk(t0, 0, first=False, prefetch=True)
        _chunk(t0 + 1, 1, first=False, prefetch=True)
        return 0

    _chunk(0, 0, first=True, prefetch=True)
    _chunk(1, 1, first=False, prefetch=True)
    lax.fori_loop(1, (NCHUNK - 1) // 2, pair, 0)
    _chunk(NCHUNK - 1, 0, first=False, prefetch=False)
    _wait(_scat_descs(0))

    w = s * NC + c
    pltpu.sync_copy(red_v, scal_out.at[w])

    plsc.subcore_barrier()

    def _wseg(t, _):
        base = pl.multiple_of(segbase + t * WB, WB)
        pltpu.sync_copy(acc_sh.at[pl.ds(base, WB)],
                        acc_out.at[c, pl.ds(base, WB)])
        return 0

    lax.fori_loop(0, SEG // WB, _wseg, 0)


def _elem_stage(tab, conn_i, conn_j, prep):
    mesh = plsc.VectorSubcoreMesh(core_axis_name="c", subcore_axis_name="s")
    f = functools.partial(
        pl.kernel,
        out_type=[
            jax.ShapeDtypeStruct((NC, NPAD, 16), f32),
            jax.ShapeDtypeStruct((NC * NS, 1, 16), f32),
        ],
        mesh=mesh,
        compiler_params=pltpu.CompilerParams(
            use_tc_tiling_on_sc=False, needs_layout_passes=False),
        scratch_types=[
            pltpu.VMEM((2, NGB, GB), i32),
            pltpu.VMEM((2, NGB, GB), i32),
            pltpu.VMEM((CH, 16), f32),
            pltpu.VMEM((CH, 16), f32),
            pltpu.VMEM((2, 15, CH), f32),
            pltpu.VMEM((ZB, 16), f32),
            pltpu.VMEM((1, 16), f32),
            pltpu.VMEM_SHARED((NPAD, 16), f32),
            pltpu.SemaphoreType.DMA,
            pltpu.SemaphoreType.DMA,
            pltpu.SemaphoreType.DMA,
            pltpu.SemaphoreType.DMA,
            pltpu.SemaphoreType.DMA,
            pltpu.SemaphoreType.DMA,
            pltpu.SemaphoreType.DMA,
            pltpu.SemaphoreType.DMA,
            pltpu.SemaphoreType.DMA,
            pltpu.SemaphoreType.DMA,
        ],
    )(_elem_body)
    return f(tab, conn_i, conn_j, *prep)


def _nreduce_body(acc_h, bcd_h, bcr_h, part_out,
                  a0_v, a1_v, bcd_v, bcr_v, red_v):
    c = lax.axis_index("c")
    s = lax.axis_index("s")
    w = s * NC + c
    lane = lax.iota(i32, 16)
    zeros16 = jnp.zeros((16,), f32)
    for k in range(8):
        red_v[k, :] = zeros16
    base0 = w * NPT

    def chunk(t, _):
        base = pl.multiple_of(base0 + t * RC, RC)
        pltpu.sync_copy(acc_h.at[0, pl.ds(base, RC)], a0_v)
        pltpu.sync_copy(acc_h.at[1, pl.ds(base, RC)], a1_v)
        pltpu.sync_copy(bcd_h.at[pl.ds(base, RC)], bcd_v)
        pltpu.sync_copy(bcr_h.at[pl.ds(base, RC)], bcr_v)

        def group(g, _):
            r = g * 16 + lane
            gs = pl.multiple_of(g * 16, 16)

            def av(k):
                kk = jnp.full((16,), k, i32)
                return (plsc.load_gather(a0_v, [r, kk])
                        + plsc.load_gather(a1_v, [r, kk]))

            v0, v1, v2 = av(0), av(1), av(2)
            m0, m1, m2c = av(3), av(4), av(5)
            e0, e1, e2 = av(6), av(7), av(8)
            bd = bcd_v[pl.ds(gs, 16)]
            br = bcr_v[pl.ds(gs, 16)]
            ones = jnp.full((16,), 1.0, f32)
            free_d = jnp.where(bd < 0.5, ones, zeros16)
            free_r = jnp.where(br < 0.5, ones, zeros16)
            pin = jnp.where(bd > 0.5, free_r, zeros16)
            t0 = v0 + e0
            t1 = v1 + e1
            t2 = v2 + e2
            fif = t0 * t0 + t1 * t1 + t2 * t2
            fe2 = e0 * e0 + e1 * e1 + e2 * e2
            m2 = m0 * m0 + m1 * m1 + m2c * m2c
            red_v[0, :] = red_v[0, :] + fif * free_d
            red_v[1, :] = red_v[1, :] + fe2 * free_d
            red_v[2, :] = red_v[2, :] + m2 * free_r
            red_v[3, :] = red_v[3, :] + m2 * pin
            red_v[4, :] = red_v[4, :] + free_d
            red_v[5, :] = red_v[5, :] + free_r
            red_v[6, :] = red_v[6, :] + pin
            return 0

        lax.fori_loop(0, RC // 16, group, 0)
        return 0

    lax.fori_loop(0, NPT // RC, chunk, 0)
    pltpu.sync_copy(red_v, part_out.at[w])


def _nreduce_stage(acc, bcd_p, bcr_p):
    mesh = plsc.VectorSubcoreMesh(core_axis_name="c", subcore_axis_name="s")
    f = functools.partial(
        pl.kernel,
        out_type=jax.ShapeDtypeStruct((NC * NS, 8, 16), f32),
        mesh=mesh,
        compiler_params=pltpu.CompilerParams(
            use_tc_tiling_on_sc=False, needs_layout_passes=False),
        scratch_types=[
            pltpu.VMEM((RC, 16), f32),
            pltpu.VMEM((RC, 16), f32),
            pltpu.VMEM((RC,), f32),
            pltpu.VMEM((RC,), f32),
            pltpu.VMEM((8, 16), f32),
        ],
    )(_nreduce_body)
    return f(acc, bcd_p, bcr_p)


def _final_body(part_ref, scal_ref, escal_ref, out_ref):
    p = part_ref[...]
    kin_tot = jnp.sum(scal_ref[...])
    s_fif = jnp.sum(p[:, 0, :])
    s_fe2 = jnp.sum(p[:, 1, :])
    s_mr = jnp.sum(p[:, 2, :])
    s_mp = jnp.sum(p[:, 3, :])
    c_d = jnp.sum(p[:, 4, :])
    c_r = jnp.sum(p[:, 5, :])
    c_p = jnp.sum(p[:, 6, :])
    l_sum = escal_ref[0]
    l_max = escal_ref[1]
    q_max = escal_ref[2]
    nd = jnp.maximum(c_d * 3.0, 1.0)
    nr = jnp.maximum(c_r * 3.0, 1.0)
    npin = jnp.maximum(c_p * 3.0, 1.0)
    F_char = jnp.maximum(jnp.sqrt(s_fe2 / nd), 1.0)
    M_char = jnp.maximum(jnp.maximum(q_max, 1.0) * l_max * l_sum / 8.0, 1.0)
    L_force = s_fif / (F_char * F_char) / nd
    L_moment = s_mr / (M_char * M_char) / nr
    L_neumann = s_mp / (M_char * M_char) / npin
    L_kin = 0.5 * kin_tot / float(E)
    total = (W_FORCE * L_force + W_MOMENT * L_moment
             + W_NEUMANN * L_neumann + W_KIN * L_kin)
    out_ref[...] = jnp.reshape(total, (1, 1))


def _final_stage(partials, scal, escal):
    return pl.pallas_call(
        _final_body,
        in_specs=[
            pl.BlockSpec((NC * NS, 8, 16), lambda: (0, 0, 0)),
            pl.BlockSpec((NC * NS, 1, 16), lambda: (0, 0, 0)),
            pl.BlockSpec(memory_space=pltpu.MemorySpace.SMEM),
        ],
        out_specs=pl.BlockSpec((1, 1), lambda: (0, 0)),
        out_shape=jax.ShapeDtypeStruct((1, 1), f32),
    )(partials, scal, escal)


def kernel(coords, conn, prop_E, prop_A, prop_I22, elem_lengths, elem_directions,
           elem_load, bc_disp, bc_rot, W1, b1, W2, b2):
    coords_p = jnp.pad(coords, ((0, NPAD - N), (0, 0)))
    node_out = _node_stage(coords_p, W1, b1, W2, b2)
    pred = node_out[0][:N]
    tab = _repack_stage(node_out[1:])
    ep = E2 - E
    prep = _prep_stage(
        jnp.pad(prop_E, (0, ep)), jnp.pad(prop_A, (0, ep)),
        jnp.pad(prop_I22, (0, ep)), jnp.pad(elem_lengths, (0, ep)),
        jnp.pad(elem_directions[:, 0], (0, ep)),
        jnp.pad(elem_directions[:, 1], (0, ep)),
        jnp.pad(elem_directions[:, 2], (0, ep)),
        jnp.pad(elem_load[:, 0], (0, ep)),
        jnp.pad(elem_load[:, 1], (0, ep)),
        jnp.pad(elem_load[:, 2], (0, ep)))
    conn_i = conn[:, 0].astype(i32)
    conn_j = conn[:, 1].astype(i32)
    acc, scal = _elem_stage(tab, conn_i, conn_j, prep[:15])
    bcd_p = jnp.pad(bc_disp[:, 0], (0, NPAD - N), constant_values=1.0)
    bcr_p = jnp.pad(bc_rot[:, 0], (0, NPAD - N), constant_values=1.0)
    partials = _nreduce_stage(acc, bcd_p, bcr_p)
    total = _final_stage(partials, scal, prep[15])
    return total.reshape(()), pred


# speedup vs baseline: 59.7672x; 1.0066x over previous
import functools

import jax
import jax.numpy as jnp
from jax import lax
from jax.experimental import pallas as pl
from jax.experimental.pallas import tpu as pltpu
from jax.experimental.pallas import tpu_sc as plsc

N = 100000
E = 1600000
H = 64

NC = 2
NS = 16
EPC = E // NC
EPT = EPC // NS
CH = 400
NCHUNK = EPT // CH
GRP = CH // 16
GB = 80
NGB = CH // GB
ZB = 112
WB = 784
SEG = 6272
NPAD = NS * SEG
NPT = NPAD // (NC * NS)
RC = 224
BE = 8192
E2 = 196 * BE
NBE = E2 // BE
BN = 1024
NBN = NPAD // BN

W_FORCE = 1.0
W_MOMENT = 1.0
W_KIN = 0.1
W_NEUMANN = 1.0

f32 = jnp.float32
i32 = jnp.int32


def _node_body(c_ref, w1_ref, b1_ref, w2_ref, b2_ref, pred_ref, *col_refs):
    c = c_ref[...]
    w1 = w1_ref[...]
    b1 = b1_ref[...]
    w2 = w2_ref[...]
    b2 = b2_ref[...]
    z = jnp.dot(c, w1, preferred_element_type=f32) + b1[None, :]
    h = jnp.tanh(z)
    pred = jnp.dot(h, w2, preferred_element_type=f32) + b2[None, :]
    s = 1.0 - h * h
    dn = (((1,), (1,)), ((), ()))
    g0 = lax.dot_general(s * w2[:, 0][None, :], w1, dn, preferred_element_type=f32)
    g1 = lax.dot_general(s * w2[:, 1][None, :], w1, dn, preferred_element_type=f32)
    g2 = lax.dot_general(s * w2[:, 2][None, :], w1, dn, preferred_element_type=f32)
    pred_ref[...] = pred
    for k in range(3):
        col_refs[k][...] = g0[:, k]
        col_refs[3 + k][...] = g1[:, k]
        col_refs[6 + k][...] = g2[:, k]
    col_refs[9][...] = pred[:, 2]


def _node_stage(coords_p, W1, b1, W2, b2):
    nv = jax.ShapeDtypeStruct((NPAD,), f32)
    return pl.pallas_call(
        _node_body,
        grid=(NBN,),
        in_specs=[
            pl.BlockSpec((BN, 3), lambda i: (i, 0)),
            pl.BlockSpec((3, H), lambda i: (0, 0)),
            pl.BlockSpec((H,), lambda i: (0,)),
            pl.BlockSpec((H, 3), lambda i: (0, 0)),
            pl.BlockSpec((3,), lambda i: (0,)),
        ],
        out_specs=[pl.BlockSpec((BN, 3), lambda i: (i, 0))]
        + [pl.BlockSpec((BN,), lambda i: (i,))] * 10,
        out_shape=[jax.ShapeDtypeStruct((NPAD, 3), f32)] + [nv] * 10,
    )(coords_p, W1, b1, W2, b2)


def _prep_body(pE_ref, pA_ref, pI_ref, L_ref, d0_ref, d1_ref, d2_ref,
               q0_ref, q1_ref, q2_ref,
               eah_ref, ei_ref, eil_ref,
               x0_ref, x1_ref, x2_ref, z0_ref, z1_ref, z2_ref,
               y0_ref, y1_ref, y2_ref, f0_ref, f1_ref, f2_ref, es_ref,
               acc_s):
    i = pl.program_id(0)
    pE = pE_ref[...]
    pA = pA_ref[...]
    pI = pI_ref[...]
    L = L_ref[...]
    d0 = d0_ref[...]
    d1 = d1_ref[...]
    d2 = d2_ref[...]
    q0 = q0_ref[...]
    q1 = q1_ref[...]
    q2 = q2_ref[...]
    par = jnp.abs(d1) > 0.99
    zero = jnp.zeros_like(d0)
    z0 = jnp.where(par, d1, -d2)
    z1 = jnp.where(par, -d0, zero)
    z2 = jnp.where(par, zero, d0)
    zn = jnp.maximum(jnp.sqrt(z0 * z0 + z1 * z1 + z2 * z2), 1e-8)
    z0, z1, z2 = z0 / zn, z1 / zn, z2 / zn
    y0 = z1 * d2 - z2 * d1
    y1 = z2 * d0 - z0 * d2
    y2 = z0 * d1 - z1 * d0
    yn = jnp.maximum(jnp.sqrt(y0 * y0 + y1 * y1 + y2 * y2), 1e-8)
    y0, y1, y2 = y0 / yn, y1 / yn, y2 / yn
    EA = pE * pA
    EI = pE * pI
    eah_ref[...] = 0.5 * EA
    ei_ref[...] = EI
    eil_ref[...] = EI / L
    x0_ref[...] = d0
    x1_ref[...] = d1
    x2_ref[...] = d2
    z0_ref[...] = z0
    z1_ref[...] = z1
    z2_ref[...] = z2
    y0_ref[...] = y0
    y1_ref[...] = y1
    y2_ref[...] = y2
    f0_ref[...] = q0 * L * 0.5
    f1_ref[...] = q1 * L * 0.5
    f2_ref[...] = q2 * L * 0.5

    @pl.when(i == 0)
    def _():
        acc_s[0] = 0.0
        acc_s[1] = 0.0
        acc_s[2] = 0.0

    acc_s[0] = acc_s[0] + jnp.sum(L)
    acc_s[1] = jnp.maximum(acc_s[1], jnp.max(L))
    qm = jnp.maximum(jnp.max(jnp.abs(q0)), jnp.max(jnp.abs(q1)))
    acc_s[2] = jnp.maximum(acc_s[2], jnp.maximum(qm, jnp.max(jnp.abs(q2))))

    @pl.when(i == NBE - 1)
    def _():
        es_ref[0] = acc_s[0]
        es_ref[1] = acc_s[1]
        es_ref[2] = acc_s[2]
        for k in range(3, 8):
            es_ref[k] = 0.0


def _prep_stage(*cols):
    ev = jax.ShapeDtypeStruct((E2,), f32)
    return pl.pallas_call(
        _prep_body,
        grid=(NBE,),
        in_specs=[pl.BlockSpec((BE,), lambda i: (i,))] * 10,
        out_specs=[pl.BlockSpec((BE,), lambda i: (i,))] * 15
        + [pl.BlockSpec(memory_space=pltpu.MemorySpace.SMEM)],
        out_shape=[ev] * 15 + [jax.ShapeDtypeStruct((8,), f32)],
        scratch_shapes=[pltpu.SMEM((8,), f32)],
    )(*cols)


def _repack_body(*refs):
    cols = refs[:10]
    tab_out = refs[10]
    in_v = refs[11]
    out_v = refs[12]
    rsem = refs[13]
    c = lax.axis_index("c")
    s = lax.axis_index("s")
    w = s * NC + c
    lane = lax.iota(i32, 16)
    zeros16 = jnp.zeros((16,), f32)

    def _zrow(r, _):
        out_v[r, :] = zeros16
        return 0

    lax.fori_loop(0, RC, _zrow, 0)
    base0 = w * NPT

    def chunk(t, _):
        base = pl.multiple_of(base0 + t * RC, RC)
        rcps = [pltpu.async_copy(cols[k].at[pl.ds(base, RC)], in_v.at[k], rsem)
                for k in range(10)]
        for cp in rcps:
            cp.wait()

        def group(g, _):
            r = g * 16 + lane
            gs = pl.multiple_of(g * 16, 16)
            for k in range(10):
                plsc.store_scatter(out_v, [r, jnp.full((16,), k, i32)],
                                   in_v[k, pl.ds(gs, 16)])
            return 0

        lax.fori_loop(0, RC // 16, group, 0)
        pltpu.sync_copy(out_v, tab_out.at[pl.ds(base, RC)])
        return 0

    lax.fori_loop(0, NPT // RC, chunk, 0)


def _repack_stage(cols):
    mesh = plsc.VectorSubcoreMesh(core_axis_name="c", subcore_axis_name="s")
    f = functools.partial(
        pl.kernel,
        out_type=jax.ShapeDtypeStruct((NPAD, 16), f32),
        mesh=mesh,
        compiler_params=pltpu.CompilerParams(
            use_tc_tiling_on_sc=False, needs_layout_passes=False),
        scratch_types=[
            pltpu.VMEM((10, RC), f32),
            pltpu.VMEM((RC, 16), f32),
            pltpu.SemaphoreType.DMA,
        ],
    )(_repack_body)
    return f(*cols)


def _elem_body(tab_hbm, ci_hbm, cj_hbm,
               eah_h, ei_h, eil_h, x0_h, x1_h, x2_h, z0_h, z1_h, z2_h,
               y0_h, y1_h, y2_h, f0_h, f1_h, f2_h,
               acc_out, scal_out,
               idx_vi, idx_vj, rows_vi, rows_vj, fld_v, zbuf, red_v,
               acc_sh, sem_c0, sem_c1, sem_f0, sem_f1, sem_s,
               sg0, sg1, sg2, sg3, sg4):
    c = lax.axis_index("c")
    s = lax.axis_index("s")
    zeros16 = jnp.zeros((16,), f32)
    lane = lax.iota(i32, 16)
    sem_c = (sem_c0, sem_c1)
    sem_f = (sem_f0, sem_f1)
    sem_g = (sg0, sg1, sg2, sg3, sg4)
    fields = (eah_h, ei_h, eil_h, x0_h, x1_h, x2_h, z0_h, z1_h, z2_h,
              y0_h, y1_h, y2_h, f0_h, f1_h, f2_h)

    def _zrow(r, _):
        zbuf[r, :] = zeros16
        return 0

    lax.fori_loop(0, ZB, _zrow, 0)
    segbase = s * SEG

    def _zseg(t, _):
        base = pl.multiple_of(segbase + t * ZB, ZB)
        pltpu.sync_copy(zbuf, acc_sh.at[pl.ds(base, ZB)])
        return 0

    lax.fori_loop(0, SEG // ZB, _zseg, 0)
    red_v[0, :] = zeros16
    plsc.subcore_barrier()

    ebase = c * EPC + s * EPT

    def _off(t):
        return pl.multiple_of(ebase + t * CH, CH)

    def _conn_descs(t, b):
        off = _off(t)
        ds_ = []
        for j in range(NGB):
            o = pl.multiple_of(off + j * GB, GB)
            ds_.append(pltpu.make_async_copy(ci_hbm.at[pl.ds(o, GB)],
                                             idx_vi.at[b, j], sem_c[b]))
            ds_.append(pltpu.make_async_copy(cj_hbm.at[pl.ds(o, GB)],
                                             idx_vj.at[b, j], sem_c[b]))
        return ds_

    def _field_descs(t, b):
        off = _off(t)
        return [pltpu.make_async_copy(fh.at[pl.ds(off, CH)],
                                      fld_v.at[b, k], sem_f[b])
                for k, fh in enumerate(fields)]

    def _scat_descs(b):
        ds_ = []
        for j in range(NGB):
            ds_.append(pltpu.make_async_copy(
                rows_vi.at[pl.ds(j * GB, GB)],
                acc_sh.at[idx_vi.at[b, j]], sem_s))
            ds_.append(pltpu.make_async_copy(
                rows_vj.at[pl.ds(j * GB, GB)],
                acc_sh.at[idx_vj.at[b, j]], sem_s))
        return ds_

    def _fire(descs):
        for d in descs:
            d.start()

    def _wait(descs):
        for d in descs:
            d.wait()

    def _compute_sub(j, b):
        def group(g, _):
            r = g * 16 + lane

            def col(k):
                return jnp.full((16,), k, i32)

            def ldi(k):
                return plsc.load_gather(rows_vi, [r, col(k)])

            def ldj(k):
                return plsc.load_gather(rows_vj, [r, col(k)])

            gs = pl.multiple_of(g * 16, 16)

            def fl(k):
                return fld_v[b, k, pl.ds(gs, 16)]

            gxi0, gxi1, gxi2 = ldi(0), ldi(1), ldi(2)
            gzi0, gzi1, gzi2 = ldi(3), ldi(4), ldi(5)
            gpi0, gpi1, gpi2 = ldi(6), ldi(7), ldi(8)
            phi_i = ldi(9)
            gxj0, gxj1, gxj2 = ldj(0), ldj(1), ldj(2)
            gzj0, gzj1, gzj2 = ldj(3), ldj(4), ldj(5)
            gpj0, gpj1, gpj2 = ldj(6), ldj(7), ldj(8)
            phi_j = ldj(9)
            EAh = fl(0)
            EIe = fl(1)
            EIL = fl(2)
            xh0 = fl(3)
            xh1 = fl(4)
            xh2 = fl(5)
            z0 = fl(6)
            z1 = fl(7)
            z2 = fl(8)
            y0 = fl(9)
            y1 = fl(10)
            y2 = fl(11)
            Fe0 = fl(12)
            Fe1 = fl(13)
            Fe2 = fl(14)

            dotxi = gxi0 * xh0 + gxi1 * xh1 + gxi2 * xh2
            dotzi = gzi0 * xh0 + gzi1 * xh1 + gzi2 * xh2
            dotxj = gxj0 * xh0 + gxj1 * xh1 + gxj2 * xh2
            dotzj = gzj0 * xh0 + gzj1 * xh1 + gzj2 * xh2
            eps_i = xh0 * dotxi + xh2 * dotzi
            eps_j = xh0 * dotxj + xh2 * dotzj
            kap_i = gpi0 * xh0 + gpi1 * xh1 + gpi2 * xh2
            kap_j = gpj0 * xh0 + gpj1 * xh1 + gpj2 * xh2
            N_avg = EAh * (eps_i + eps_j)
            M_i = EIe * kap_i
            M_j = EIe * kap_j
            V = EIL * (kap_j - kap_i)
            Fi0 = N_avg * xh0 + V * z0
            Fi1 = N_avg * xh1 + V * z1
            Fi2 = N_avg * xh2 + V * z2

            du_i = z0 * dotxi + z2 * dotzi
            du_j = z0 * dotxj + z2 * dotzj
            rk_i = phi_i - du_i
            rk_j = phi_j - du_j
            red_v[0, :] = red_v[0, :] + rk_i * rk_i + rk_j * rk_j

            def sti(k, v):
                plsc.store_scatter(rows_vi, [r, col(k)], v)

            def stj(k, v):
                plsc.store_scatter(rows_vj, [r, col(k)], v)

            sti(0, Fi0)
            sti(1, Fi1)
            sti(2, Fi2)
            sti(3, M_i * y0)
            sti(4, M_i * y1)
            sti(5, M_i * y2)
            sti(6, Fe0)
            sti(7, Fe1)
            sti(8, Fe2)
            stj(0, -Fi0)
            stj(1, -Fi1)
            stj(2, -Fi2)
            stj(3, M_j * y0)
            stj(4, M_j * y1)
            stj(5, M_j * y2)
            stj(6, Fe0)
            stj(7, Fe1)
            stj(8, Fe2)
            return 0

        lax.fori_loop(j * (GRP // NGB), (j + 1) * (GRP // NGB), group, 0)

    def _chunk(t, b, first, prefetch):
        _wait(_conn_descs(t, b))
        if not first:
            _wait(_scat_descs(1 - b))
        gds = []
        for j in range(NGB):
            gds.append(pltpu.make_async_copy(
                tab_hbm.at[idx_vi.at[b, j]],
                rows_vi.at[pl.ds(j * GB, GB)], sem_g[j]))
            gds.append(pltpu.make_async_copy(
                tab_hbm.at[idx_vj.at[b, j]],
                rows_vj.at[pl.ds(j * GB, GB)], sem_g[j]))
        _fire(gds)
        if prefetch:
            tn = t + 1
            _fire(_conn_descs(tn, 1 - b))
            _fire(_field_descs(tn, 1 - b))
        _wait(_field_descs(t, b))
        for j in range(NGB):
            gds[2 * j].wait()
            gds[2 * j + 1].wait()
            _compute_sub(j, b)
            sd_i = pltpu.make_async_copy(
                rows_vi.at[pl.ds(j * GB, GB)],
                acc_sh.at[idx_vi.at[b, j]], sem_s)
            sd_j = pltpu.make_async_copy(
                rows_vj.at[pl.ds(j * GB, GB)],
                acc_sh.at[idx_vj.at[b, j]], sem_s)
            sd_i.start(add=True)
            sd_j.start(add=True)

    _fire(_conn_descs(0, 0))
    _fire(_field_descs(0, 0))

    def pair(u, _):
        t0 = u * 2
        _chunk(t0, 0, first=False, prefetch=True)
        _chunk(t0 + 1, 1, first=False, prefetch=True)
        return 0

    _chunk(0, 0, first=True, prefetch=True)
    _chunk(1, 1, first=False, prefetch=True)
    lax.fori_loop(1, (NCHUNK - 1) // 2, pair, 0)
    _chunk(NCHUNK - 1, 0, first=False, prefetch=False)
    _wait(_scat_descs(0))

    w = s * NC + c
    pltpu.sync_copy(red_v, scal_out.at[w])

    plsc.subcore_barrier()

    def _wseg(t, _):
        base = pl.multiple_of(segbase + t * WB, WB)
        pltpu.sync_copy(acc_sh.at[pl.ds(base, WB)],
                        acc_out.at[c, pl.ds(base, WB)])
        return 0

    lax.fori_loop(0, SEG // WB, _wseg, 0)


def _elem_stage(tab, conn_i, conn_j, prep):
    mesh = plsc.VectorSubcoreMesh(core_axis_name="c", subcore_axis_name="s")
    f = functools.partial(
        pl.kernel,
        out_type=[
            jax.ShapeDtypeStruct((NC, NPAD, 16), f32),
            jax.ShapeDtypeStruct((NC * NS, 1, 16), f32),
        ],
        mesh=mesh,
        compiler_params=pltpu.CompilerParams(
            use_tc_tiling_on_sc=False, needs_layout_passes=False),
        scratch_types=[
            pltpu.VMEM((2, NGB, GB), i32),
            pltpu.VMEM((2, NGB, GB), i32),
            pltpu.VMEM((CH, 16), f32),
            pltpu.VMEM((CH, 16), f32),
            pltpu.VMEM((2, 15, CH), f32),
            pltpu.VMEM((ZB, 16), f32),
            pltpu.VMEM((1, 16), f32),
            pltpu.VMEM_SHARED((NPAD, 16), f32),
            pltpu.SemaphoreType.DMA,
            pltpu.SemaphoreType.DMA,
            pltpu.SemaphoreType.DMA,
            pltpu.SemaphoreType.DMA,
            pltpu.SemaphoreType.DMA,
            pltpu.SemaphoreType.DMA,
            pltpu.SemaphoreType.DMA,
            pltpu.SemaphoreType.DMA,
            pltpu.SemaphoreType.DMA,
            pltpu.SemaphoreType.DMA,
        ],
    )(_elem_body)
    return f(tab, conn_i, conn_j, *prep)


def _nreduce_body(acc_h, bcd_h, bcr_h, part_out,
                  a0_v, a1_v, bcd_v, bcr_v, red_v, nsem):
    c = lax.axis_index("c")
    s = lax.axis_index("s")
    w = s * NC + c
    lane = lax.iota(i32, 16)
    zeros16 = jnp.zeros((16,), f32)
    for k in range(8):
        red_v[k, :] = zeros16
    base0 = w * NPT

    def chunk(t, _):
        base = pl.multiple_of(base0 + t * RC, RC)
        ncps = [pltpu.async_copy(acc_h.at[0, pl.ds(base, RC)], a0_v, nsem),
                pltpu.async_copy(acc_h.at[1, pl.ds(base, RC)], a1_v, nsem),
                pltpu.async_copy(bcd_h.at[pl.ds(base, RC)], bcd_v, nsem),
                pltpu.async_copy(bcr_h.at[pl.ds(base, RC)], bcr_v, nsem)]
        for cp in ncps:
            cp.wait()

        def group(g, _):
            r = g * 16 + lane
            gs = pl.multiple_of(g * 16, 16)

            def av(k):
                kk = jnp.full((16,), k, i32)
                return (plsc.load_gather(a0_v, [r, kk])
                        + plsc.load_gather(a1_v, [r, kk]))

            v0, v1, v2 = av(0), av(1), av(2)
            m0, m1, m2c = av(3), av(4), av(5)
            e0, e1, e2 = av(6), av(7), av(8)
            bd = bcd_v[pl.ds(gs, 16)]
            br = bcr_v[pl.ds(gs, 16)]
            ones = jnp.full((16,), 1.0, f32)
            free_d = jnp.where(bd < 0.5, ones, zeros16)
            free_r = jnp.where(br < 0.5, ones, zeros16)
            pin = jnp.where(bd > 0.5, free_r, zeros16)
            t0 = v0 + e0
            t1 = v1 + e1
            t2 = v2 + e2
            fif = t0 * t0 + t1 * t1 + t2 * t2
            fe2 = e0 * e0 + e1 * e1 + e2 * e2
            m2 = m0 * m0 + m1 * m1 + m2c * m2c
            red_v[0, :] = red_v[0, :] + fif * free_d
            red_v[1, :] = red_v[1, :] + fe2 * free_d
            red_v[2, :] = red_v[2, :] + m2 * free_r
            red_v[3, :] = red_v[3, :] + m2 * pin
            red_v[4, :] = red_v[4, :] + free_d
            red_v[5, :] = red_v[5, :] + free_r
            red_v[6, :] = red_v[6, :] + pin
            return 0

        lax.fori_loop(0, RC // 16, group, 0)
        return 0

    lax.fori_loop(0, NPT // RC, chunk, 0)
    pltpu.sync_copy(red_v, part_out.at[w])


def _nreduce_stage(acc, bcd_p, bcr_p):
    mesh = plsc.VectorSubcoreMesh(core_axis_name="c", subcore_axis_name="s")
    f = functools.partial(
        pl.kernel,
        out_type=jax.ShapeDtypeStruct((NC * NS, 8, 16), f32),
        mesh=mesh,
        compiler_params=pltpu.CompilerParams(
            use_tc_tiling_on_sc=False, needs_layout_passes=False),
        scratch_types=[
            pltpu.VMEM((RC, 16), f32),
            pltpu.VMEM((RC, 16), f32),
            pltpu.VMEM((RC,), f32),
            pltpu.VMEM((RC,), f32),
            pltpu.VMEM((8, 16), f32),
            pltpu.SemaphoreType.DMA,
        ],
    )(_nreduce_body)
    return f(acc, bcd_p, bcr_p)


def _final_body(part_ref, scal_ref, escal_ref, out_ref):
    p = part_ref[...]
    kin_tot = jnp.sum(scal_ref[...])
    s_fif = jnp.sum(p[:, 0, :])
    s_fe2 = jnp.sum(p[:, 1, :])
    s_mr = jnp.sum(p[:, 2, :])
    s_mp = jnp.sum(p[:, 3, :])
    c_d = jnp.sum(p[:, 4, :])
    c_r = jnp.sum(p[:, 5, :])
    c_p = jnp.sum(p[:, 6, :])
    l_sum = escal_ref[0]
    l_max = escal_ref[1]
    q_max = escal_ref[2]
    nd = jnp.maximum(c_d * 3.0, 1.0)
    nr = jnp.maximum(c_r * 3.0, 1.0)
    npin = jnp.maximum(c_p * 3.0, 1.0)
    F_char = jnp.maximum(jnp.sqrt(s_fe2 / nd), 1.0)
    M_char = jnp.maximum(jnp.maximum(q_max, 1.0) * l_max * l_sum / 8.0, 1.0)
    L_force = s_fif / (F_char * F_char) / nd
    L_moment = s_mr / (M_char * M_char) / nr
    L_neumann = s_mp / (M_char * M_char) / npin
    L_kin = 0.5 * kin_tot / float(E)
    total = (W_FORCE * L_force + W_MOMENT * L_moment
             + W_NEUMANN * L_neumann + W_KIN * L_kin)
    out_ref[...] = jnp.reshape(total, (1, 1))


def _final_stage(partials, scal, escal):
    return pl.pallas_call(
        _final_body,
        in_specs=[
            pl.BlockSpec((NC * NS, 8, 16), lambda: (0, 0, 0)),
            pl.BlockSpec((NC * NS, 1, 16), lambda: (0, 0, 0)),
            pl.BlockSpec(memory_space=pltpu.MemorySpace.SMEM),
        ],
        out_specs=pl.BlockSpec((1, 1), lambda: (0, 0)),
        out_shape=jax.ShapeDtypeStruct((1, 1), f32),
    )(partials, scal, escal)


def kernel(coords, conn, prop_E, prop_A, prop_I22, elem_lengths, elem_directions,
           elem_load, bc_disp, bc_rot, W1, b1, W2, b2):
    coords_p = jnp.pad(coords, ((0, NPAD - N), (0, 0)))
    node_out = _node_stage(coords_p, W1, b1, W2, b2)
    pred = node_out[0][:N]
    tab = _repack_stage(node_out[1:])
    ep = E2 - E
    prep = _prep_stage(
        jnp.pad(prop_E, (0, ep)), jnp.pad(prop_A, (0, ep)),
        jnp.pad(prop_I22, (0, ep)), jnp.pad(elem_lengths, (0, ep)),
        jnp.pad(elem_directions[:, 0], (0, ep)),
        jnp.pad(elem_directions[:, 1], (0, ep)),
        jnp.pad(elem_directions[:, 2], (0, ep)),
        jnp.pad(elem_load[:, 0], (0, ep)),
        jnp.pad(elem_load[:, 1], (0, ep)),
        jnp.pad(elem_load[:, 2], (0, ep)))
    conn_i = conn[:, 0].astype(i32)
    conn_j = conn[:, 1].astype(i32)
    acc, scal = _elem_stage(tab, conn_i, conn_j, prep[:15])
    bcd_p = jnp.pad(bc_disp[:, 0], (0, NPAD - N), constant_values=1.0)
    bcr_p = jnp.pad(bc_rot[:, 0], (0, NPAD - N), constant_values=1.0)
    partials = _nreduce_stage(acc, bcd_p, bcr_p)
    total = _final_stage(partials, scal, prep[15])
    return total.reshape(()), pred


# speedup vs baseline: 60.0888x; 1.0054x over previous
import functools

import jax
import jax.numpy as jnp
from jax import lax
from jax.experimental import pallas as pl
from jax.experimental.pallas import tpu as pltpu
from jax.experimental.pallas import tpu_sc as plsc

N = 100000
E = 1600000
H = 64

NC = 2
NS = 16
EPC = E // NC
EPT = EPC // NS
CH = 400
NCHUNK = EPT // CH
GRP = CH // 16
GB = 80
NGB = CH // GB
ZB = 112
WB = 784
SEG = 6272
NPAD = NS * SEG
NPT = NPAD // (NC * NS)
RC = 224
BE = 8192
E2 = 196 * BE
NBE = E2 // BE
BN = 1024
NBN = NPAD // BN

W_FORCE = 1.0
W_MOMENT = 1.0
W_KIN = 0.1
W_NEUMANN = 1.0

f32 = jnp.float32
i32 = jnp.int32


def _node_body(c_ref, w1_ref, b1_ref, w2_ref, b2_ref, pred_ref, *col_refs):
    c = c_ref[...]
    w1 = w1_ref[...]
    b1 = b1_ref[...]
    w2 = w2_ref[...]
    b2 = b2_ref[...]
    z = jnp.dot(c, w1, preferred_element_type=f32) + b1[None, :]
    h = jnp.tanh(z)
    pred = jnp.dot(h, w2, preferred_element_type=f32) + b2[None, :]
    s = 1.0 - h * h
    dn = (((1,), (1,)), ((), ()))
    g0 = lax.dot_general(s * w2[:, 0][None, :], w1, dn, preferred_element_type=f32)
    g1 = lax.dot_general(s * w2[:, 1][None, :], w1, dn, preferred_element_type=f32)
    g2 = lax.dot_general(s * w2[:, 2][None, :], w1, dn, preferred_element_type=f32)
    pred_ref[...] = pred
    for k in range(3):
        col_refs[k][...] = g0[:, k]
        col_refs[3 + k][...] = g1[:, k]
        col_refs[6 + k][...] = g2[:, k]
    col_refs[9][...] = pred[:, 2]


def _node_stage(coords_p, W1, b1, W2, b2):
    nv = jax.ShapeDtypeStruct((NPAD,), f32)
    return pl.pallas_call(
        _node_body,
        grid=(NBN,),
        in_specs=[
            pl.BlockSpec((BN, 3), lambda i: (i, 0)),
            pl.BlockSpec((3, H), lambda i: (0, 0)),
            pl.BlockSpec((H,), lambda i: (0,)),
            pl.BlockSpec((H, 3), lambda i: (0, 0)),
            pl.BlockSpec((3,), lambda i: (0,)),
        ],
        out_specs=[pl.BlockSpec((BN, 3), lambda i: (i, 0))]
        + [pl.BlockSpec((BN,), lambda i: (i,))] * 10,
        out_shape=[jax.ShapeDtypeStruct((NPAD, 3), f32)] + [nv] * 10,
    )(coords_p, W1, b1, W2, b2)


def _prep_body(pE_ref, pA_ref, pI_ref, L_ref, d0_ref, d1_ref, d2_ref,
               q0_ref, q1_ref, q2_ref,
               eah_ref, ei_ref, eil_ref,
               x0_ref, x1_ref, x2_ref, z0_ref, z1_ref, z2_ref,
               y0_ref, y1_ref, y2_ref, f0_ref, f1_ref, f2_ref, es_ref,
               acc_s):
    i = pl.program_id(0)
    pE = pE_ref[...]
    pA = pA_ref[...]
    pI = pI_ref[...]
    L = L_ref[...]
    d0 = d0_ref[...]
    d1 = d1_ref[...]
    d2 = d2_ref[...]
    q0 = q0_ref[...]
    q1 = q1_ref[...]
    q2 = q2_ref[...]
    par = jnp.abs(d1) > 0.99
    zero = jnp.zeros_like(d0)
    z0 = jnp.where(par, d1, -d2)
    z1 = jnp.where(par, -d0, zero)
    z2 = jnp.where(par, zero, d0)
    zn = jnp.maximum(jnp.sqrt(z0 * z0 + z1 * z1 + z2 * z2), 1e-8)
    z0, z1, z2 = z0 / zn, z1 / zn, z2 / zn
    y0 = z1 * d2 - z2 * d1
    y1 = z2 * d0 - z0 * d2
    y2 = z0 * d1 - z1 * d0
    yn = jnp.maximum(jnp.sqrt(y0 * y0 + y1 * y1 + y2 * y2), 1e-8)
    y0, y1, y2 = y0 / yn, y1 / yn, y2 / yn
    EA = pE * pA
    EI = pE * pI
    eah_ref[...] = 0.5 * EA
    ei_ref[...] = EI
    eil_ref[...] = EI / L
    x0_ref[...] = d0
    x1_ref[...] = d1
    x2_ref[...] = d2
    z0_ref[...] = z0
    z1_ref[...] = z1
    z2_ref[...] = z2
    y0_ref[...] = y0
    y1_ref[...] = y1
    y2_ref[...] = y2
    f0_ref[...] = q0 * L * 0.5
    f1_ref[...] = q1 * L * 0.5
    f2_ref[...] = q2 * L * 0.5

    @pl.when(i == 0)
    def _():
        acc_s[0] = 0.0
        acc_s[1] = 0.0
        acc_s[2] = 0.0

    valid = (i * BE + lax.broadcasted_iota(i32, (BE,), 0)) < E
    zv = jnp.zeros((BE,), f32)
    L_m = jnp.where(valid, L, zv)
    acc_s[0] = acc_s[0] + jnp.sum(L_m)
    acc_s[1] = jnp.maximum(acc_s[1], jnp.max(L_m))
    qm = jnp.maximum(jnp.max(jnp.where(valid, jnp.abs(q0), zv)),
                     jnp.max(jnp.where(valid, jnp.abs(q1), zv)))
    acc_s[2] = jnp.maximum(acc_s[2], jnp.maximum(
        qm, jnp.max(jnp.where(valid, jnp.abs(q2), zv))))

    @pl.when(i == NBE - 1)
    def _():
        es_ref[0] = acc_s[0]
        es_ref[1] = acc_s[1]
        es_ref[2] = acc_s[2]
        for k in range(3, 8):
            es_ref[k] = 0.0


def _prep_stage(*cols):
    ev = jax.ShapeDtypeStruct((E,), f32)
    return pl.pallas_call(
        _prep_body,
        grid=(NBE,),
        in_specs=[pl.BlockSpec((BE,), lambda i: (i,))] * 10,
        out_specs=[pl.BlockSpec((BE,), lambda i: (i,))] * 15
        + [pl.BlockSpec(memory_space=pltpu.MemorySpace.SMEM)],
        out_shape=[ev] * 15 + [jax.ShapeDtypeStruct((8,), f32)],
        scratch_shapes=[pltpu.SMEM((8,), f32)],
    )(*cols)


def _repack_body(*refs):
    cols = refs[:10]
    tab_out = refs[10]
    in_v = refs[11]
    out_v = refs[12]
    rsem = refs[13]
    c = lax.axis_index("c")
    s = lax.axis_index("s")
    w = s * NC + c
    lane = lax.iota(i32, 16)
    zeros16 = jnp.zeros((16,), f32)

    def _zrow(r, _):
        out_v[r, :] = zeros16
        return 0

    lax.fori_loop(0, RC, _zrow, 0)
    base0 = w * NPT

    def chunk(t, _):
        base = pl.multiple_of(base0 + t * RC, RC)
        rcps = [pltpu.async_copy(cols[k].at[pl.ds(base, RC)], in_v.at[k], rsem)
                for k in range(10)]
        for cp in rcps:
            cp.wait()

        def group(g, _):
            r = g * 16 + lane
            gs = pl.multiple_of(g * 16, 16)
            for k in range(10):
                plsc.store_scatter(out_v, [r, jnp.full((16,), k, i32)],
                                   in_v[k, pl.ds(gs, 16)])
            return 0

        lax.fori_loop(0, RC // 16, group, 0)
        pltpu.sync_copy(out_v, tab_out.at[pl.ds(base, RC)])
        return 0

    lax.fori_loop(0, NPT // RC, chunk, 0)


def _repack_stage(cols):
    mesh = plsc.VectorSubcoreMesh(core_axis_name="c", subcore_axis_name="s")
    f = functools.partial(
        pl.kernel,
        out_type=jax.ShapeDtypeStruct((NPAD, 16), f32),
        mesh=mesh,
        compiler_params=pltpu.CompilerParams(
            use_tc_tiling_on_sc=False, needs_layout_passes=False),
        scratch_types=[
            pltpu.VMEM((10, RC), f32),
            pltpu.VMEM((RC, 16), f32),
            pltpu.SemaphoreType.DMA,
        ],
    )(_repack_body)
    return f(*cols)


def _elem_body(tab_hbm, ci_hbm, cj_hbm,
               eah_h, ei_h, eil_h, x0_h, x1_h, x2_h, z0_h, z1_h, z2_h,
               y0_h, y1_h, y2_h, f0_h, f1_h, f2_h,
               acc_out, scal_out,
               idx_vi, idx_vj, rows_vi, rows_vj, fld_v, zbuf, red_v,
               acc_sh, sem_c0, sem_c1, sem_f0, sem_f1, sem_s,
               sg0, sg1, sg2, sg3, sg4):
    c = lax.axis_index("c")
    s = lax.axis_index("s")
    zeros16 = jnp.zeros((16,), f32)
    lane = lax.iota(i32, 16)
    sem_c = (sem_c0, sem_c1)
    sem_f = (sem_f0, sem_f1)
    sem_g = (sg0, sg1, sg2, sg3, sg4)
    fields = (eah_h, ei_h, eil_h, x0_h, x1_h, x2_h, z0_h, z1_h, z2_h,
              y0_h, y1_h, y2_h, f0_h, f1_h, f2_h)

    def _zrow(r, _):
        zbuf[r, :] = zeros16
        return 0

    lax.fori_loop(0, ZB, _zrow, 0)
    segbase = s * SEG

    def _zseg(t, _):
        base = pl.multiple_of(segbase + t * ZB, ZB)
        pltpu.sync_copy(zbuf, acc_sh.at[pl.ds(base, ZB)])
        return 0

    lax.fori_loop(0, SEG // ZB, _zseg, 0)
    red_v[0, :] = zeros16
    plsc.subcore_barrier()

    ebase = c * EPC + s * EPT

    def _off(t):
        return pl.multiple_of(ebase + t * CH, CH)

    def _conn_descs(t, b):
        off = _off(t)
        ds_ = []
        for j in range(NGB):
            o = pl.multiple_of(off + j * GB, GB)
            ds_.append(pltpu.make_async_copy(ci_hbm.at[pl.ds(o, GB)],
                                             idx_vi.at[b, j], sem_c[b]))
            ds_.append(pltpu.make_async_copy(cj_hbm.at[pl.ds(o, GB)],
                                             idx_vj.at[b, j], sem_c[b]))
        return ds_

    def _field_descs(t, b):
        off = _off(t)
        return [pltpu.make_async_copy(fh.at[pl.ds(off, CH)],
                                      fld_v.at[b, k], sem_f[b])
                for k, fh in enumerate(fields)]

    def _scat_descs(b):
        ds_ = []
        for j in range(NGB):
            ds_.append(pltpu.make_async_copy(
                rows_vi.at[pl.ds(j * GB, GB)],
                acc_sh.at[idx_vi.at[b, j]], sem_s))
            ds_.append(pltpu.make_async_copy(
                rows_vj.at[pl.ds(j * GB, GB)],
                acc_sh.at[idx_vj.at[b, j]], sem_s))
        return ds_

    def _fire(descs):
        for d in descs:
            d.start()

    def _wait(descs):
        for d in descs:
            d.wait()

    def _compute_sub(j, b):
        def group(g, _):
            r = g * 16 + lane

            def col(k):
                return jnp.full((16,), k, i32)

            def ldi(k):
                return plsc.load_gather(rows_vi, [r, col(k)])

            def ldj(k):
                return plsc.load_gather(rows_vj, [r, col(k)])

            gs = pl.multiple_of(g * 16, 16)

            def fl(k):
                return fld_v[b, k, pl.ds(gs, 16)]

            gxi0, gxi1, gxi2 = ldi(0), ldi(1), ldi(2)
            gzi0, gzi1, gzi2 = ldi(3), ldi(4), ldi(5)
            gpi0, gpi1, gpi2 = ldi(6), ldi(7), ldi(8)
            phi_i = ldi(9)
            gxj0, gxj1, gxj2 = ldj(0), ldj(1), ldj(2)
            gzj0, gzj1, gzj2 = ldj(3), ldj(4), ldj(5)
            gpj0, gpj1, gpj2 = ldj(6), ldj(7), ldj(8)
            phi_j = ldj(9)
            EAh = fl(0)
            EIe = fl(1)
            EIL = fl(2)
            xh0 = fl(3)
            xh1 = fl(4)
            xh2 = fl(5)
            z0 = fl(6)
            z1 = fl(7)
            z2 = fl(8)
            y0 = fl(9)
            y1 = fl(10)
            y2 = fl(11)
            Fe0 = fl(12)
            Fe1 = fl(13)
            Fe2 = fl(14)

            dotxi = gxi0 * xh0 + gxi1 * xh1 + gxi2 * xh2
            dotzi = gzi0 * xh0 + gzi1 * xh1 + gzi2 * xh2
            dotxj = gxj0 * xh0 + gxj1 * xh1 + gxj2 * xh2
            dotzj = gzj0 * xh0 + gzj1 * xh1 + gzj2 * xh2
            eps_i = xh0 * dotxi + xh2 * dotzi
            eps_j = xh0 * dotxj + xh2 * dotzj
            kap_i = gpi0 * xh0 + gpi1 * xh1 + gpi2 * xh2
            kap_j = gpj0 * xh0 + gpj1 * xh1 + gpj2 * xh2
            N_avg = EAh * (eps_i + eps_j)
            M_i = EIe * kap_i
            M_j = EIe * kap_j
            V = EIL * (kap_j - kap_i)
            Fi0 = N_avg * xh0 + V * z0
            Fi1 = N_avg * xh1 + V * z1
            Fi2 = N_avg * xh2 + V * z2

            du_i = z0 * dotxi + z2 * dotzi
            du_j = z0 * dotxj + z2 * dotzj
            rk_i = phi_i - du_i
            rk_j = phi_j - du_j
            red_v[0, :] = red_v[0, :] + rk_i * rk_i + rk_j * rk_j

            def sti(k, v):
                plsc.store_scatter(rows_vi, [r, col(k)], v)

            def stj(k, v):
                plsc.store_scatter(rows_vj, [r, col(k)], v)

            sti(0, Fi0)
            sti(1, Fi1)
            sti(2, Fi2)
            sti(3, M_i * y0)
            sti(4, M_i * y1)
            sti(5, M_i * y2)
            sti(6, Fe0)
            sti(7, Fe1)
            sti(8, Fe2)
            stj(0, -Fi0)
            stj(1, -Fi1)
            stj(2, -Fi2)
            stj(3, M_j * y0)
            stj(4, M_j * y1)
            stj(5, M_j * y2)
            stj(6, Fe0)
            stj(7, Fe1)
            stj(8, Fe2)
            return 0

        lax.fori_loop(j * (GRP // NGB), (j + 1) * (GRP // NGB), group, 0)

    def _chunk(t, b, first, prefetch):
        _wait(_conn_descs(t, b))
        if not first:
            _wait(_scat_descs(1 - b))
        gds = []
        for j in range(NGB):
            gds.append(pltpu.make_async_copy(
                tab_hbm.at[idx_vi.at[b, j]],
                rows_vi.at[pl.ds(j * GB, GB)], sem_g[j]))
            gds.append(pltpu.make_async_copy(
                tab_hbm.at[idx_vj.at[b, j]],
                rows_vj.at[pl.ds(j * GB, GB)], sem_g[j]))
        _fire(gds)
        if prefetch:
            tn = t + 1
            _fire(_conn_descs(tn, 1 - b))
            _fire(_field_descs(tn, 1 - b))
        _wait(_field_descs(t, b))
        for j in range(NGB):
            gds[2 * j].wait()
            gds[2 * j + 1].wait()
            _compute_sub(j, b)
            sd_i = pltpu.make_async_copy(
                rows_vi.at[pl.ds(j * GB, GB)],
                acc_sh.at[idx_vi.at[b, j]], sem_s)
            sd_j = pltpu.make_async_copy(
                rows_vj.at[pl.ds(j * GB, GB)],
                acc_sh.at[idx_vj.at[b, j]], sem_s)
            sd_i.start(add=True)
            sd_j.start(add=True)

    _fire(_conn_descs(0, 0))
    _fire(_field_descs(0, 0))

    def pair(u, _):
        t0 = u * 2
        _chunk(t0, 0, first=False, prefetch=True)
        _chunk(t0 + 1, 1, first=False, prefetch=True)
        return 0

    _chunk(0, 0, first=True, prefetch=True)
    _chunk(1, 1, first=False, prefetch=True)
    lax.fori_loop(1, (NCHUNK - 1) // 2, pair, 0)
    _chunk(NCHUNK - 1, 0, first=False, prefetch=False)
    _wait(_scat_descs(0))

    w = s * NC + c
    pltpu.sync_copy(red_v, scal_out.at[w])

    plsc.subcore_barrier()

    def _wseg(t, _):
        base = pl.multiple_of(segbase + t * WB, WB)
        pltpu.sync_copy(acc_sh.at[pl.ds(base, WB)],
                        acc_out.at[c, pl.ds(base, WB)])
        return 0

    lax.fori_loop(0, SEG // WB, _wseg, 0)


def _elem_stage(tab, conn_i, conn_j, prep):
    mesh = plsc.VectorSubcoreMesh(core_axis_name="c", subcore_axis_name="s")
    f = functools.partial(
        pl.kernel,
        out_type=[
            jax.ShapeDtypeStruct((NC, NPAD, 16), f32),
            jax.ShapeDtypeStruct((NC * NS, 1, 16), f32),
        ],
        mesh=mesh,
        compiler_params=pltpu.CompilerParams(
            use_tc_tiling_on_sc=False, needs_layout_passes=False),
        scratch_types=[
            pltpu.VMEM((2, NGB, GB), i32),
            pltpu.VMEM((2, NGB, GB), i32),
            pltpu.VMEM((CH, 16), f32),
            pltpu.VMEM((CH, 16), f32),
            pltpu.VMEM((2, 15, CH), f32),
            pltpu.VMEM((ZB, 16), f32),
            pltpu.VMEM((1, 16), f32),
            pltpu.VMEM_SHARED((NPAD, 16), f32),
            pltpu.SemaphoreType.DMA,
            pltpu.SemaphoreType.DMA,
            pltpu.SemaphoreType.DMA,
            pltpu.SemaphoreType.DMA,
            pltpu.SemaphoreType.DMA,
            pltpu.SemaphoreType.DMA,
            pltpu.SemaphoreType.DMA,
            pltpu.SemaphoreType.DMA,
            pltpu.SemaphoreType.DMA,
            pltpu.SemaphoreType.DMA,
        ],
    )(_elem_body)
    return f(tab, conn_i, conn_j, *prep)


def _nreduce_body(acc_h, bcd_h, bcr_h, part_out,
                  a0_v, a1_v, bcd_v, bcr_v, red_v, nsem):
    c = lax.axis_index("c")
    s = lax.axis_index("s")
    w = s * NC + c
    lane = lax.iota(i32, 16)
    zeros16 = jnp.zeros((16,), f32)
    for k in range(8):
        red_v[k, :] = zeros16
    base0 = w * NPT

    def chunk(t, _):
        base = pl.multiple_of(base0 + t * RC, RC)
        ncps = [pltpu.async_copy(acc_h.at[0, pl.ds(base, RC)], a0_v, nsem),
                pltpu.async_copy(acc_h.at[1, pl.ds(base, RC)], a1_v, nsem),
                pltpu.async_copy(bcd_h.at[pl.ds(base, RC)], bcd_v, nsem),
                pltpu.async_copy(bcr_h.at[pl.ds(base, RC)], bcr_v, nsem)]
        for cp in ncps:
            cp.wait()

        def group(g, _):
            r = g * 16 + lane
            gs = pl.multiple_of(g * 16, 16)

            def av(k):
                kk = jnp.full((16,), k, i32)
                return (plsc.load_gather(a0_v, [r, kk])
                        + plsc.load_gather(a1_v, [r, kk]))

            v0, v1, v2 = av(0), av(1), av(2)
            m0, m1, m2c = av(3), av(4), av(5)
            e0, e1, e2 = av(6), av(7), av(8)
            bd = bcd_v[pl.ds(gs, 16)]
            br = bcr_v[pl.ds(gs, 16)]
            ones = jnp.full((16,), 1.0, f32)
            free_d = jnp.where(bd < 0.5, ones, zeros16)
            free_r = jnp.where(br < 0.5, ones, zeros16)
            pin = jnp.where(bd > 0.5, free_r, zeros16)
            t0 = v0 + e0
            t1 = v1 + e1
            t2 = v2 + e2
            fif = t0 * t0 + t1 * t1 + t2 * t2
            fe2 = e0 * e0 + e1 * e1 + e2 * e2
            m2 = m0 * m0 + m1 * m1 + m2c * m2c
            red_v[0, :] = red_v[0, :] + fif * free_d
            red_v[1, :] = red_v[1, :] + fe2 * free_d
            red_v[2, :] = red_v[2, :] + m2 * free_r
            red_v[3, :] = red_v[3, :] + m2 * pin
            red_v[4, :] = red_v[4, :] + free_d
            red_v[5, :] = red_v[5, :] + free_r
            red_v[6, :] = red_v[6, :] + pin
            return 0

        lax.fori_loop(0, RC // 16, group, 0)
        return 0

    lax.fori_loop(0, NPT // RC, chunk, 0)
    pltpu.sync_copy(red_v, part_out.at[w])


def _nreduce_stage(acc, bcd_p, bcr_p):
    mesh = plsc.VectorSubcoreMesh(core_axis_name="c", subcore_axis_name="s")
    f = functools.partial(
        pl.kernel,
        out_type=jax.ShapeDtypeStruct((NC * NS, 8, 16), f32),
        mesh=mesh,
        compiler_params=pltpu.CompilerParams(
            use_tc_tiling_on_sc=False, needs_layout_passes=False),
        scratch_types=[
            pltpu.VMEM((RC, 16), f32),
            pltpu.VMEM((RC, 16), f32),
            pltpu.VMEM((RC,), f32),
            pltpu.VMEM((RC,), f32),
            pltpu.VMEM((8, 16), f32),
            pltpu.SemaphoreType.DMA,
        ],
    )(_nreduce_body)
    return f(acc, bcd_p, bcr_p)


def _final_body(part_ref, scal_ref, escal_ref, out_ref):
    p = part_ref[...]
    kin_tot = jnp.sum(scal_ref[...])
    s_fif = jnp.sum(p[:, 0, :])
    s_fe2 = jnp.sum(p[:, 1, :])
    s_mr = jnp.sum(p[:, 2, :])
    s_mp = jnp.sum(p[:, 3, :])
    c_d = jnp.sum(p[:, 4, :])
    c_r = jnp.sum(p[:, 5, :])
    c_p = jnp.sum(p[:, 6, :])
    l_sum = escal_ref[0]
    l_max = escal_ref[1]
    q_max = escal_ref[2]
    nd = jnp.maximum(c_d * 3.0, 1.0)
    nr = jnp.maximum(c_r * 3.0, 1.0)
    npin = jnp.maximum(c_p * 3.0, 1.0)
    F_char = jnp.maximum(jnp.sqrt(s_fe2 / nd), 1.0)
    M_char = jnp.maximum(jnp.maximum(q_max, 1.0) * l_max * l_sum / 8.0, 1.0)
    L_force = s_fif / (F_char * F_char) / nd
    L_moment = s_mr / (M_char * M_char) / nr
    L_neumann = s_mp / (M_char * M_char) / npin
    L_kin = 0.5 * kin_tot / float(E)
    total = (W_FORCE * L_force + W_MOMENT * L_moment
             + W_NEUMANN * L_neumann + W_KIN * L_kin)
    out_ref[...] = jnp.reshape(total, (1, 1))


def _final_stage(partials, scal, escal):
    return pl.pallas_call(
        _final_body,
        in_specs=[
            pl.BlockSpec((NC * NS, 8, 16), lambda: (0, 0, 0)),
            pl.BlockSpec((NC * NS, 1, 16), lambda: (0, 0, 0)),
            pl.BlockSpec(memory_space=pltpu.MemorySpace.SMEM),
        ],
        out_specs=pl.BlockSpec((1, 1), lambda: (0, 0)),
        out_shape=jax.ShapeDtypeStruct((1, 1), f32),
    )(partials, scal, escal)


def kernel(coords, conn, prop_E, prop_A, prop_I22, elem_lengths, elem_directions,
           elem_load, bc_disp, bc_rot, W1, b1, W2, b2):
    coords_p = jnp.pad(coords, ((0, NPAD - N), (0, 0)))
    node_out = _node_stage(coords_p, W1, b1, W2, b2)
    pred = node_out[0][:N]
    tab = _repack_stage(node_out[1:])
    prep = _prep_stage(
        prop_E, prop_A, prop_I22, elem_lengths,
        elem_directions[:, 0], elem_directions[:, 1], elem_directions[:, 2],
        elem_load[:, 0], elem_load[:, 1], elem_load[:, 2])
    conn_i = conn[:, 0].astype(i32)
    conn_j = conn[:, 1].astype(i32)
    acc, scal = _elem_stage(tab, conn_i, conn_j, prep[:15])
    bcd_p = jnp.pad(bc_disp[:, 0], (0, NPAD - N), constant_values=1.0)
    bcr_p = jnp.pad(bc_rot[:, 0], (0, NPAD - N), constant_values=1.0)
    partials = _nreduce_stage(acc, bcd_p, bcr_p)
    total = _final_stage(partials, scal, prep[15])
    return total.reshape(()), pred
